# Initial kernel scaffold; baseline (speedup 1.0000x reference)
import jax
import jax.numpy as jnp
from jax.experimental import pallas as pl


def kernel(x_list, edge_index_list, Wq, bq, Wk, bk, Wv, bv, ln_g, ln_b, W1, b1, W2, b2):
    raise NotImplementedError("write your pallas kernel here")


# speedup vs baseline: 3.0843x; 3.0843x over previous
import functools
import math

import jax
import jax.numpy as jnp
import numpy as np
from jax import lax
from jax.experimental import pallas as pl
from jax.experimental.pallas import tpu as pltpu
from jax.experimental.pallas import tpu_sc as plsc

T, N, E = 3, 10000, 320000
D_IN, HID, H = 128, 128, 8
DK = HID // H
HG = 4
HW = HG * DK
NS = 16
NP = 10240
CW = 128
ECH = E // CW
QB = [(tt, ts) for tt in range(T) for ts in range(tt + 1)]
TS_OF = [ts for (_, ts) in QB]
F32 = jnp.float32
I32 = jnp.int32
_SQRT2 = math.sqrt(2.0)


def _qb0(tt):
    return tt * (tt + 1) // 2


_BLK = 2000


def _proj_body(x_ref, wq_ref, wk_ref, wv_ref, bq_ref, bk_ref, bv_ref,
               q_ref, k_ref, v_ref):
    x = x_ref[...]
    q_ref[...] = jnp.dot(x, wq_ref[...], preferred_element_type=F32) + bq_ref[...]
    k_ref[...] = jnp.dot(x, wk_ref[...], preferred_element_type=F32) + bk_ref[...]
    v_ref[...] = jnp.dot(x, wv_ref[...], preferred_element_type=F32) + bv_ref[...]


_proj = pl.pallas_call(
    _proj_body,
    grid=(T * N // _BLK,),
    in_specs=[pl.BlockSpec((_BLK, D_IN), lambda i: (i, 0))]
    + [pl.BlockSpec((D_IN, HID), lambda i: (0, 0))] * 3
    + [pl.BlockSpec((1, HID), lambda i: (0, 0))] * 3,
    out_specs=[pl.BlockSpec((_BLK, HID), lambda i: (i, 0))] * 3,
    out_shape=[jax.ShapeDtypeStruct((T * N, HID), F32)] * 3,
)


def _att_body(qe_ref, ke_ref, s_ref, e1_ref):
    p = qe_ref[...] * ke_ref[...]
    att = jnp.dot(p, s_ref[...].reshape(HID, HG), preferred_element_type=F32)
    e1_ref[...] = jnp.exp(att * 0.25)


_att = pl.pallas_call(
    _att_body,
    grid=(2, 6, E // _BLK),
    in_specs=[
        pl.BlockSpec((_BLK, HID), lambda c, b, i: (b * (E // _BLK) + i, 0)),
        pl.BlockSpec((_BLK, HID),
                     lambda c, b, i: (
                         (b - jnp.where(b >= 3, 3, jnp.where(b >= 1, 1, 0)))
                         * (E // _BLK) + i, 0)),
        pl.BlockSpec((1, HID, HG), lambda c, b, i: (c, 0, 0)),
    ],
    out_specs=pl.BlockSpec((_BLK, HG),
                           lambda c, b, i: ((c * 6 + b) * (E // _BLK) + i, 0)),
    out_shape=jax.ShapeDtypeStruct((12 * E, HG), F32),
)


def _ffn_body(y_ref, g_ref, b_ref, w1_ref, b1_ref, w2_ref, b2_ref, o_ref):
    y = y_ref[...]
    mu = jnp.mean(y, axis=-1, keepdims=True)
    var = jnp.mean((y - mu) ** 2, axis=-1, keepdims=True)
    hn = (y - mu) / jnp.sqrt(var + 1e-5) * g_ref[...] + b_ref[...]
    h1 = jnp.dot(hn, w1_ref[...], preferred_element_type=F32) + b1_ref[...]
    h1 = 0.5 * h1 * (1.0 + lax.erf(h1 / _SQRT2))
    h2 = jnp.dot(h1, w2_ref[...], preferred_element_type=F32) + b2_ref[...]
    o_ref[...] = y + h2


_ffn = pl.pallas_call(
    _ffn_body,
    grid=(2 * T * N // _BLK,),
    in_specs=[pl.BlockSpec((_BLK, HID), lambda i: (i, 0)),
              pl.BlockSpec((1, HID), lambda i: (0, 0)),
              pl.BlockSpec((1, HID), lambda i: (0, 0)),
              pl.BlockSpec((HID, 2 * HID), lambda i: (0, 0)),
              pl.BlockSpec((1, 2 * HID), lambda i: (0, 0)),
              pl.BlockSpec((2 * HID, HID), lambda i: (0, 0)),
              pl.BlockSpec((1, HID), lambda i: (0, 0))],
    out_specs=pl.BlockSpec((_BLK, HID), lambda i: (i, 0)),
    out_shape=jax.ShapeDtypeStruct((2 * T * N, HID), F32),
)


def _mesh():
    return plsc.VectorSubcoreMesh(core_axis_name="c", subcore_axis_name="s")


def _span(s, total):
    base, extra = total // NS, total % NS
    lo = s * base + jnp.minimum(s, extra)
    cnt = base + jnp.where(s < extra, 1, 0)
    return lo, lo + cnt


def _iota16():
    return lax.broadcasted_iota(I32, (16,), 0)


def _spread2d(dst_ref, flat_ref, off, nrow):
    for j in range(nrow):
        for i in range(CW // 16):
            w = flat_ref[pl.ds(j * CW + i * 16, 16)]
            if off is not None:
                w = w + off
            dst_ref[j, pl.ds(i * 16, 16)] = w


def _scatter_rows16(vals_flat_ref, rows_ref, nquad):
    iota = _iota16()
    perms = [(j * 4 + iota) & 15 for j in range(4)]
    lt4 = iota < 4

    def q_body(q, cr):
        win = vals_flat_ref[pl.ds(q * 16, 16)]
        for j in range(4):
            row = jnp.where(lt4, jnp.take(win, perms[j]), 0.0)
            rows_ref[q * 4 + j, :] = row
        return cr

    lax.fori_loop(0, nquad, q_body, 0)


def _merge_den(den_rows_ref, q):
    iota = _iota16()
    h_of = iota & 3
    m = None
    for j in range(4):
        w = jnp.take(den_rows_ref[4 * q + j, :], h_of)
        m = w if m is None else jnp.where((iota >> 2) == j, w, m)
    return m


_AM = 1280
_AMC = E // _AM
_AK = _AM // CW
_GM = 512
_GMC = E // _GM
_GK = _GM // CW


def _make_gatherA():
    out_type = (jax.ShapeDtypeStruct((6 * E, HID), F32),
                jax.ShapeDtypeStruct((3 * E, HID), F32),
                jax.ShapeDtypeStruct((3 * E, HID), F32))
    scratch = [
        pltpu.VMEM((_GM,), I32),
        pltpu.VMEM((_GK, CW), I32),
        pltpu.VMEM((_GM, HID), F32),
        pltpu.SemaphoreType.DMA,
    ]

    @functools.partial(
        pl.kernel, out_type=out_type, mesh=_mesh(), scratch_types=scratch,
        compiler_params=pltpu.CompilerParams(use_tc_tiling_on_sc=False))
    def gatherA(qtab, ktab, vtab, edges1d, qe_hbm, ke_hbm, ve_hbm,
                dflat, iv, rows, sem):
        c = lax.axis_index("c")
        s = lax.axis_index("s")
        wid = s * 2 + c
        base, extra = _GMC // 32, _GMC % 32
        lo = wid * base + jnp.minimum(wid, extra)
        hi = lo + base + jnp.where(wid < extra, 1, 0)

        jobs = []
        for b, (tt, ts) in enumerate(QB):
            jobs.append((qtab, tt, 1, ts, qe_hbm, b))
        for ts in range(T):
            jobs.append((ktab, ts, 0, ts, ke_hbm, ts))
            jobs.append((vtab, ts, 0, ts, ve_hbm, ts))

        for tab, toff, rsel, ts, out, ob in jobs:
            def macro(m, cr, tab=tab, toff=toff, rsel=rsel, ts=ts, out=out,
                      ob=ob):
                eoff = (ts * 2 + rsel) * E + m * _GM
                pltpu.sync_copy(edges1d.at[pl.ds(eoff, _GM)], dflat)
                _spread2d(iv, dflat, toff * N, _GK)
                descs = []
                for j in range(_GK):
                    descs.append(pltpu.async_copy(
                        tab.at[iv.at[j]], rows.at[pl.ds(j * CW, CW)], sem))
                for d in descs:
                    d.wait()
                pltpu.sync_copy(
                    rows, out.at[pl.ds(ob * E + m * _GM, _GM)])
                return cr

            lax.fori_loop(lo, hi, macro, 0)

    return gatherA


def _make_passC(t_tar):
    nsrc = t_tar + 1
    out_type = jax.ShapeDtypeStruct((2 * NP, 16), F32)
    scratch = [
        pltpu.VMEM((_AM,), I32),
        pltpu.VMEM((_AK, CW), I32),
        pltpu.VMEM((_AM * HG,), F32),
        pltpu.VMEM((_AM, 16), F32),
        pltpu.VMEM_SHARED((NP, 16), F32),
        pltpu.SemaphoreType.DMA,
    ]

    @functools.partial(
        pl.kernel, out_type=out_type, mesh=_mesh(), scratch_types=scratch,
        compiler_params=pltpu.CompilerParams(use_tc_tiling_on_sc=False))
    def passC(edges1d, e1_hbm, zden, den_hbm, dflat, dstv, e1f, vrows,
              sh_den, sem):
        c = lax.axis_index("c")
        s = lax.axis_index("s")

        @pl.when(s < 8)
        def _():
            pltpu.sync_copy(zden, sh_den.at[pl.ds(s * 1280, 1280)])

        plsc.subcore_barrier()

        lo, hi = _span(s, _AMC)
        for ts in range(nsrc):
            e1base = (c * 6 + _qb0(t_tar) + ts) * E * HG

            def macro(m, cr, ts=ts, e1base=e1base):
                eoff = (ts * 2 + 1) * E + m * _AM
                pltpu.sync_copy(edges1d.at[pl.ds(eoff, _AM)], dflat)
                _spread2d(dstv, dflat, None, _AK)
                pltpu.sync_copy(
                    e1_hbm.at[pl.ds(e1base + m * _AM * HG, _AM * HG)], e1f)
                _scatter_rows16(e1f, vrows, _AM // 4)
                for j in range(_AK):
                    pltpu.sync_copy(vrows.at[pl.ds(j * CW, CW)],
                                    sh_den.at[dstv.at[j]], add=True)
                return cr

            lax.fori_loop(lo, hi, macro, 0)

        plsc.subcore_barrier()

        @pl.when(s < 8)
        def _():
            pltpu.sync_copy(sh_den.at[pl.ds(s * 1280, 1280)],
                            den_hbm.at[pl.ds(c * NP + s * 1280, 1280)])

    return passC


def _make_passD(t_tar):
    nsrc = t_tar + 1
    out_type = (jax.ShapeDtypeStruct((2 * nsrc * E * HG,), F32),
                jax.ShapeDtypeStruct((2 * NP, 16), F32))
    scratch = [
        pltpu.VMEM((_AM,), I32),
        pltpu.VMEM((_AK, CW), I32),
        pltpu.VMEM((_AK, CW), I32),
        pltpu.VMEM((_AM * HG,), F32),
        pltpu.VMEM((_AM * HG,), F32),
        pltpu.VMEM((_AM, 16), F32),
        pltpu.VMEM((_AM, 16), F32),
        pltpu.VMEM_SHARED((NP, 16), F32),
        pltpu.SemaphoreType.DMA,
    ]

    @functools.partial(
        pl.kernel, out_type=out_type, mesh=_mesh(), scratch_types=scratch,
        compiler_params=pltpu.CompilerParams(use_tc_tiling_on_sc=False))
    def passD(edges1d, e1_hbm, den_hbm, zden, ra_hbm, den2_hbm,
              dflat, dstv, div, e1f, e2f, denrows, vrows, sh_den2, sem):
        c = lax.axis_index("c")
        s = lax.axis_index("s")

        @pl.when(s < 8)
        def _():
            pltpu.sync_copy(zden, sh_den2.at[pl.ds(s * 1280, 1280)])

        plsc.subcore_barrier()

        lo, hi = _span(s, _AMC)
        for ts in range(nsrc):
            e1base = (c * 6 + _qb0(t_tar) + ts) * E * HG
            rabase = (c * nsrc + ts) * E * HG

            def macro(m, cr, ts=ts, e1base=e1base, rabase=rabase):
                eoff = (ts * 2 + 1) * E + m * _AM
                pltpu.sync_copy(edges1d.at[pl.ds(eoff, _AM)], dflat)
                _spread2d(dstv, dflat, None, _AK)
                _spread2d(div, dflat, c * NP, _AK)
                pltpu.sync_copy(
                    e1_hbm.at[pl.ds(e1base + m * _AM * HG, _AM * HG)], e1f)
                descs = []
                for j in range(_AK):
                    descs.append(pltpu.async_copy(
                        den_hbm.at[div.at[j]],
                        denrows.at[pl.ds(j * CW, CW)], sem))
                for d in descs:
                    d.wait()

                def quad(q, qcr):
                    sl = pl.ds(q * 16, 16)
                    den16 = _merge_den(denrows, q)
                    ra = e1f[sl] / (den16 + 1e-16)
                    e1f[sl] = ra
                    e2f[sl] = jnp.exp(-ra)
                    return qcr

                lax.fori_loop(0, _AM // 4, quad, 0)
                pltpu.sync_copy(
                    e1f, ra_hbm.at[pl.ds(rabase + m * _AM * HG, _AM * HG)])
                _scatter_rows16(e2f, vrows, _AM // 4)
                for j in range(_AK):
                    pltpu.sync_copy(vrows.at[pl.ds(j * CW, CW)],
                                    sh_den2.at[dstv.at[j]], add=True)
                return cr

            lax.fori_loop(lo, hi, macro, 0)

        plsc.subcore_barrier()

        @pl.when(s < 8)
        def _():
            pltpu.sync_copy(sh_den2.at[pl.ds(s * 1280, 1280)],
                            den2_hbm.at[pl.ds(c * NP + s * 1280, 1280)])

    return passD


_EM = 128
_EMC = E // _EM
_EK = _EM // CW


def _make_passE(t_tar):
    nsrc = t_tar + 1
    out_type = jax.ShapeDtypeStruct((2 * NP, HID), F32)
    scratch = [
        pltpu.VMEM((_EM,), I32),
        pltpu.VMEM((_EK, CW), I32),
        pltpu.VMEM((_EK, CW), I32),
        pltpu.VMEM((_EM * HG,), F32),
        pltpu.VMEM((_EM * HG,), F32),
        pltpu.VMEM((_EM, 16), F32),
        pltpu.VMEM((_EM, HID), F32),
        pltpu.VMEM((CW, HID), F32),
        pltpu.VMEM_SHARED((NP, HID), F32),
        pltpu.SemaphoreType.DMA,
    ]

    @functools.partial(
        pl.kernel, out_type=out_type, mesh=_mesh(), scratch_types=scratch,
        compiler_params=pltpu.CompilerParams(use_tc_tiling_on_sc=False))
    def passE(edges1d, ve_hbm, ra_hbm, den2_hbm, zacc, agg_hbm,
              dflat, dstv, div, raf, w2f, den2rows, ve, wv, sh_acc, sem):
        c = lax.axis_index("c")
        s = lax.axis_index("s")

        @pl.when(s < 5)
        def _():
            pltpu.sync_copy(zacc, sh_acc.at[pl.ds(s * 2048, 2048)])

        plsc.subcore_barrier()

        iota = _iota16()
        cidx = [(iota * 0) + kk for kk in range(16)]
        lo, hi = _span(s, _EMC)
        for ts in range(nsrc):
            rabase = (c * nsrc + ts) * E * HG

            def macro(m, cr, ts=ts, rabase=rabase):
                eoff = (ts * 2 + 1) * E + m * _EM
                pltpu.sync_copy(edges1d.at[pl.ds(eoff, _EM)], dflat)
                _spread2d(dstv, dflat, None, _EK)
                _spread2d(div, dflat, c * NP, _EK)
                pltpu.sync_copy(
                    ve_hbm.at[pl.ds(ts * E + m * _EM, _EM)], ve)
                pltpu.sync_copy(
                    ra_hbm.at[pl.ds(rabase + m * _EM * HG, _EM * HG)], raf)
                descs = []
                for j in range(_EK):
                    descs.append(pltpu.async_copy(
                        den2_hbm.at[div.at[j]],
                        den2rows.at[pl.ds(j * CW, CW)], sem))
                for d in descs:
                    d.wait()

                def quad(q, qcr):
                    sl = pl.ds(q * 16, 16)
                    den16 = _merge_den(den2rows, q)
                    w2f[sl] = jnp.exp(-raf[sl]) / (den16 + 1e-16)
                    return qcr

                lax.fori_loop(0, _EM // 4, quad, 0)

                for j in range(_EK):
                    def quad2(q2, qcr, j=j):
                        qq = j * (CW // 4) + q2
                        rwin = raf[pl.ds(qq * 16, 16)]
                        wwin = w2f[pl.ds(qq * 16, 16)]
                        for j2 in range(4):
                            e = qq * 4 + j2
                            el = q2 * 4 + j2
                            for h in range(HG):
                                vv = ve[e, pl.ds(c * HW + h * DK, DK)]
                                w1s = jnp.take(rwin, cidx[j2 * 4 + h])
                                w2s = jnp.take(wwin, cidx[j2 * 4 + h])
                                wv[el, pl.ds(h * DK, DK)] = vv * w1s
                                wv[el, pl.ds(HW + h * DK, DK)] = vv * w2s
                        return qcr

                    lax.fori_loop(0, CW // 4, quad2, 0)
                    pltpu.sync_copy(wv, sh_acc.at[dstv.at[j]], add=True)
                return cr

            lax.fori_loop(lo, hi, macro, 0)

        plsc.subcore_barrier()

        @pl.when(s < 5)
        def _():
            pltpu.sync_copy(sh_acc.at[pl.ds(s * 2048, 2048)],
                            agg_hbm.at[pl.ds(c * NP + s * 2048, 2048)])

    return passE


_gatherA = _make_gatherA()
_passC = [_make_passC(t) for t in range(T)]
_passD = [_make_passD(t) for t in range(T)]
_passE = [_make_passE(t) for t in range(T)]

_SEL = np.zeros((2, HID, HG), dtype=np.float32)
for _h in range(H):
    _SEL[_h // HG, _h * DK:(_h + 1) * DK, _h % HG] = 1.0


def kernel(x_list, edge_index_list, Wq, bq, Wk, bk, Wv, bv,
           ln_g, ln_b, W1, b1, W2, b2):
    xf = x_list.reshape(T * N, D_IN)
    q, k, v = _proj(xf, Wq, Wk, Wv,
                    bq.reshape(1, HID), bk.reshape(1, HID), bv.reshape(1, HID))

    qtab, ktab, vtab = q, k, v
    edges1d = edge_index_list.reshape(T * 2 * E)

    qe, ke, ve = _gatherA(qtab, ktab, vtab, edges1d)
    e1 = _att(qe, ke, jnp.asarray(_SEL)).reshape(12 * E * HG)

    zden = jnp.zeros((1280, 16), F32)
    zacc = jnp.zeros((2048, HID), F32)

    ys = []
    for t_tar in range(T):
        den = _passC[t_tar](edges1d, e1, zden)
        ra, den2 = _passD[t_tar](edges1d, e1, den, zden)
        agg = _passE[t_tar](edges1d, ve, ra, den2, zacc)
        a0, a1 = agg[:N], agg[NP:NP + N]
        causal_hat = jnp.concatenate([a0[:, :HW], a1[:, :HW]], axis=1)
        spurious_hat = jnp.concatenate([a0[:, HW:], a1[:, HW:]], axis=1)
        ys.append(causal_hat + x_list[t_tar])
        ys.append(spurious_hat)

    y = jnp.stack(ys).reshape(2 * T * N, HID)
    f = _ffn(y, ln_g.reshape(1, HID), ln_b.reshape(1, HID),
             W1, b1.reshape(1, 2 * HID), W2, b2.reshape(1, HID))
    f = f.reshape(T, 2, N, HID)
    cs, ss = f[:, 0], f[:, 1]
    return cs + ss, cs, ss


# speedup vs baseline: 3.4586x; 1.1214x over previous
import functools
import math

import jax
import jax.numpy as jnp
import numpy as np
from jax import lax
from jax.experimental import pallas as pl
from jax.experimental.pallas import tpu as pltpu
from jax.experimental.pallas import tpu_sc as plsc

T, N, E = 3, 10000, 320000
D_IN, HID, H = 128, 128, 8
DK = HID // H
HG = 4
HW = HG * DK
NS = 16
NP = 10240
CW = 128
ECH = E // CW
QB = [(tt, ts) for tt in range(T) for ts in range(tt + 1)]
TS_OF = [ts for (_, ts) in QB]
F32 = jnp.float32
I32 = jnp.int32
_SQRT2 = math.sqrt(2.0)


def _qb0(tt):
    return tt * (tt + 1) // 2


_BLK = 2000


def _proj_body(x_ref, wq_ref, wk_ref, wv_ref, bq_ref, bk_ref, bv_ref,
               q_ref, k_ref, v_ref):
    x = x_ref[...]
    q_ref[...] = jnp.dot(x, wq_ref[...], preferred_element_type=F32) + bq_ref[...]
    k_ref[...] = jnp.dot(x, wk_ref[...], preferred_element_type=F32) + bk_ref[...]
    v_ref[...] = jnp.dot(x, wv_ref[...], preferred_element_type=F32) + bv_ref[...]


_proj = pl.pallas_call(
    _proj_body,
    grid=(T * N // _BLK,),
    in_specs=[pl.BlockSpec((_BLK, D_IN), lambda i: (i, 0))]
    + [pl.BlockSpec((D_IN, HID), lambda i: (0, 0))] * 3
    + [pl.BlockSpec((1, HID), lambda i: (0, 0))] * 3,
    out_specs=[pl.BlockSpec((_BLK, HID), lambda i: (i, 0))] * 3,
    out_shape=[jax.ShapeDtypeStruct((T * N, HID), F32)] * 3,
)


def _att_body(qe_ref, ke_ref, s_ref, e1_ref):
    p = qe_ref[...] * ke_ref[...]
    att = jnp.dot(p, s_ref[...].reshape(HID, HG), preferred_element_type=F32)
    e1_ref[...] = jnp.exp(att * 0.25)


_att = pl.pallas_call(
    _att_body,
    grid=(2, 6, E // _BLK),
    in_specs=[
        pl.BlockSpec((_BLK, HID), lambda c, b, i: (b * (E // _BLK) + i, 0)),
        pl.BlockSpec((_BLK, HID),
                     lambda c, b, i: (
                         (b - jnp.where(b >= 3, 3, jnp.where(b >= 1, 1, 0)))
                         * (E // _BLK) + i, 0)),
        pl.BlockSpec((1, HID, HG), lambda c, b, i: (c, 0, 0)),
    ],
    out_specs=pl.BlockSpec((_BLK, HG),
                           lambda c, b, i: ((c * 6 + b) * (E // _BLK) + i, 0)),
    out_shape=jax.ShapeDtypeStruct((12 * E, HG), F32),
)


def _ffn_body(y_ref, g_ref, b_ref, w1_ref, b1_ref, w2_ref, b2_ref, o_ref):
    y = y_ref[...]
    mu = jnp.mean(y, axis=-1, keepdims=True)
    var = jnp.mean((y - mu) ** 2, axis=-1, keepdims=True)
    hn = (y - mu) / jnp.sqrt(var + 1e-5) * g_ref[...] + b_ref[...]
    h1 = jnp.dot(hn, w1_ref[...], preferred_element_type=F32) + b1_ref[...]
    h1 = 0.5 * h1 * (1.0 + lax.erf(h1 / _SQRT2))
    h2 = jnp.dot(h1, w2_ref[...], preferred_element_type=F32) + b2_ref[...]
    o_ref[...] = y + h2


_ffn = pl.pallas_call(
    _ffn_body,
    grid=(2 * T * N // _BLK,),
    in_specs=[pl.BlockSpec((_BLK, HID), lambda i: (i, 0)),
              pl.BlockSpec((1, HID), lambda i: (0, 0)),
              pl.BlockSpec((1, HID), lambda i: (0, 0)),
              pl.BlockSpec((HID, 2 * HID), lambda i: (0, 0)),
              pl.BlockSpec((1, 2 * HID), lambda i: (0, 0)),
              pl.BlockSpec((2 * HID, HID), lambda i: (0, 0)),
              pl.BlockSpec((1, HID), lambda i: (0, 0))],
    out_specs=pl.BlockSpec((_BLK, HID), lambda i: (i, 0)),
    out_shape=jax.ShapeDtypeStruct((2 * T * N, HID), F32),
)


def _mesh():
    return plsc.VectorSubcoreMesh(core_axis_name="c", subcore_axis_name="s")


def _span(s, total):
    base, extra = total // NS, total % NS
    lo = s * base + jnp.minimum(s, extra)
    cnt = base + jnp.where(s < extra, 1, 0)
    return lo, lo + cnt


def _iota16():
    return lax.broadcasted_iota(I32, (16,), 0)


def _spread2d(dst_ref, flat_ref, off, nrow):
    for j in range(nrow):
        for i in range(CW // 16):
            w = flat_ref[pl.ds(j * CW + i * 16, 16)]
            if off is not None:
                w = w + off
            dst_ref[j, pl.ds(i * 16, 16)] = w


def _scatter_rows16(vals_flat_ref, rows_ref, nquad):
    iota = _iota16()
    perms = [(j * 4 + iota) & 15 for j in range(4)]
    lt4 = iota < 4

    def q_body(q, cr):
        win = vals_flat_ref[pl.ds(q * 16, 16)]
        for j in range(4):
            row = jnp.where(lt4, jnp.take(win, perms[j]), 0.0)
            rows_ref[q * 4 + j, :] = row
        return cr

    lax.fori_loop(0, nquad, q_body, 0)


def _merge_den(den_rows_ref, q):
    iota = _iota16()
    h_of = iota & 3
    m = None
    for j in range(4):
        w = jnp.take(den_rows_ref[4 * q + j, :], h_of)
        m = w if m is None else jnp.where((iota >> 2) == j, w, m)
    return m


_AM = 1280
_AMC = E // _AM
_AK = _AM // CW
_GM = 512
_GMC = E // _GM
_GK = _GM // CW


def _make_gatherA():
    out_type = (jax.ShapeDtypeStruct((6 * E, HID), F32),
                jax.ShapeDtypeStruct((3 * E, HID), F32),
                jax.ShapeDtypeStruct((3 * E, HID), F32))
    scratch = [
        pltpu.VMEM((_GM,), I32),
        pltpu.VMEM((_GK, CW), I32),
        pltpu.VMEM((_GM, HID), F32),
        pltpu.SemaphoreType.DMA,
    ]

    @functools.partial(
        pl.kernel, out_type=out_type, mesh=_mesh(), scratch_types=scratch,
        compiler_params=pltpu.CompilerParams(use_tc_tiling_on_sc=False))
    def gatherA(qtab, ktab, vtab, edges1d, qe_hbm, ke_hbm, ve_hbm,
                dflat, iv, rows, sem):
        c = lax.axis_index("c")
        s = lax.axis_index("s")
        wid = s * 2 + c
        base, extra = _GMC // 32, _GMC % 32
        lo = wid * base + jnp.minimum(wid, extra)
        hi = lo + base + jnp.where(wid < extra, 1, 0)

        jobs = []
        for b, (tt, ts) in enumerate(QB):
            jobs.append((qtab, tt, 1, ts, qe_hbm, b))
        for ts in range(T):
            jobs.append((ktab, ts, 0, ts, ke_hbm, ts))
            jobs.append((vtab, ts, 0, ts, ve_hbm, ts))

        for tab, toff, rsel, ts, out, ob in jobs:
            def macro(m, cr, tab=tab, toff=toff, rsel=rsel, ts=ts, out=out,
                      ob=ob):
                eoff = (ts * 2 + rsel) * E + m * _GM
                pltpu.sync_copy(edges1d.at[pl.ds(eoff, _GM)], dflat)
                _spread2d(iv, dflat, toff * N, _GK)
                descs = []
                for j in range(_GK):
                    descs.append(pltpu.async_copy(
                        tab.at[iv.at[j]], rows.at[pl.ds(j * CW, CW)], sem))
                for d in descs:
                    d.wait()
                pltpu.sync_copy(
                    rows, out.at[pl.ds(ob * E + m * _GM, _GM)])
                return cr

            lax.fori_loop(lo, hi, macro, 0)

    return gatherA


def _make_passC(t_tar):
    nsrc = t_tar + 1
    out_type = jax.ShapeDtypeStruct((2 * NP, 16), F32)
    scratch = [
        pltpu.VMEM((_AM,), I32),
        pltpu.VMEM((_AK, CW), I32),
        pltpu.VMEM((_AM * HG,), F32),
        pltpu.VMEM((_AM, 16), F32),
        pltpu.VMEM_SHARED((NP, 16), F32),
        pltpu.SemaphoreType.DMA,
    ]

    @functools.partial(
        pl.kernel, out_type=out_type, mesh=_mesh(), scratch_types=scratch,
        compiler_params=pltpu.CompilerParams(use_tc_tiling_on_sc=False))
    def passC(edges1d, e1_hbm, zden, den_hbm, dflat, dstv, e1f, vrows,
              sh_den, sem):
        c = lax.axis_index("c")
        s = lax.axis_index("s")

        @pl.when(s < 8)
        def _():
            pltpu.sync_copy(zden, sh_den.at[pl.ds(s * 1280, 1280)])

        plsc.subcore_barrier()

        lo, hi = _span(s, _AMC)
        for ts in range(nsrc):
            e1base = (c * 6 + _qb0(t_tar) + ts) * E * HG

            def macro(m, cr, ts=ts, e1base=e1base):
                eoff = (ts * 2 + 1) * E + m * _AM
                pltpu.sync_copy(edges1d.at[pl.ds(eoff, _AM)], dflat)
                _spread2d(dstv, dflat, None, _AK)
                pltpu.sync_copy(
                    e1_hbm.at[pl.ds(e1base + m * _AM * HG, _AM * HG)], e1f)
                _scatter_rows16(e1f, vrows, _AM // 4)
                descs = []
                for j in range(_AK):
                    descs.append(pltpu.async_copy(
                        vrows.at[pl.ds(j * CW, CW)],
                        sh_den.at[dstv.at[j]], sem, add=True))
                for d in descs:
                    d.wait()
                return cr

            lax.fori_loop(lo, hi, macro, 0)

        plsc.subcore_barrier()

        @pl.when(s < 8)
        def _():
            pltpu.sync_copy(sh_den.at[pl.ds(s * 1280, 1280)],
                            den_hbm.at[pl.ds(c * NP + s * 1280, 1280)])

    return passC


def _make_passD(t_tar):
    nsrc = t_tar + 1
    out_type = (jax.ShapeDtypeStruct((2 * nsrc * E * HG,), F32),
                jax.ShapeDtypeStruct((2 * NP, 16), F32))
    scratch = [
        pltpu.VMEM((_AM,), I32),
        pltpu.VMEM((_AK, CW), I32),
        pltpu.VMEM((_AK, CW), I32),
        pltpu.VMEM((_AM * HG,), F32),
        pltpu.VMEM((_AM * HG,), F32),
        pltpu.VMEM((_AM, 16), F32),
        pltpu.VMEM((_AM, 16), F32),
        pltpu.VMEM_SHARED((NP, 16), F32),
        pltpu.SemaphoreType.DMA,
    ]

    @functools.partial(
        pl.kernel, out_type=out_type, mesh=_mesh(), scratch_types=scratch,
        compiler_params=pltpu.CompilerParams(use_tc_tiling_on_sc=False))
    def passD(edges1d, e1_hbm, den_hbm, zden, ra_hbm, den2_hbm,
              dflat, dstv, div, e1f, e2f, denrows, vrows, sh_den2, sem):
        c = lax.axis_index("c")
        s = lax.axis_index("s")

        @pl.when(s < 8)
        def _():
            pltpu.sync_copy(zden, sh_den2.at[pl.ds(s * 1280, 1280)])

        plsc.subcore_barrier()

        lo, hi = _span(s, _AMC)
        for ts in range(nsrc):
            e1base = (c * 6 + _qb0(t_tar) + ts) * E * HG
            rabase = (c * nsrc + ts) * E * HG

            def macro(m, cr, ts=ts, e1base=e1base, rabase=rabase):
                eoff = (ts * 2 + 1) * E + m * _AM
                pltpu.sync_copy(edges1d.at[pl.ds(eoff, _AM)], dflat)
                _spread2d(dstv, dflat, None, _AK)
                _spread2d(div, dflat, c * NP, _AK)
                pltpu.sync_copy(
                    e1_hbm.at[pl.ds(e1base + m * _AM * HG, _AM * HG)], e1f)
                descs = []
                for j in range(_AK):
                    descs.append(pltpu.async_copy(
                        den_hbm.at[div.at[j]],
                        denrows.at[pl.ds(j * CW, CW)], sem))
                for d in descs:
                    d.wait()

                def quad(q, qcr):
                    sl = pl.ds(q * 16, 16)
                    den16 = _merge_den(denrows, q)
                    ra = e1f[sl] / (den16 + 1e-16)
                    e1f[sl] = ra
                    e2f[sl] = jnp.exp(-ra)
                    return qcr

                lax.fori_loop(0, _AM // 4, quad, 0)
                pltpu.sync_copy(
                    e1f, ra_hbm.at[pl.ds(rabase + m * _AM * HG, _AM * HG)])
                _scatter_rows16(e2f, vrows, _AM // 4)
                descs2 = []
                for j in range(_AK):
                    descs2.append(pltpu.async_copy(
                        vrows.at[pl.ds(j * CW, CW)],
                        sh_den2.at[dstv.at[j]], sem, add=True))
                for d in descs2:
                    d.wait()
                return cr

            lax.fori_loop(lo, hi, macro, 0)

        plsc.subcore_barrier()

        @pl.when(s < 8)
        def _():
            pltpu.sync_copy(sh_den2.at[pl.ds(s * 1280, 1280)],
                            den2_hbm.at[pl.ds(c * NP + s * 1280, 1280)])

    return passD


_EM = 128
_EMC = E // _EM
_EK = _EM // CW


def _make_passE(t_tar):
    nsrc = t_tar + 1
    out_type = jax.ShapeDtypeStruct((2 * NP, HID), F32)
    scratch = [
        pltpu.VMEM((2, _EM), I32),
        pltpu.VMEM((1, CW), I32),
        pltpu.VMEM((1, CW), I32),
        pltpu.VMEM((_EM * HG,), F32),
        pltpu.VMEM((_EM * HG,), F32),
        pltpu.VMEM((_EM, 16), F32),
        pltpu.VMEM((_EM, HID), F32),
        pltpu.VMEM((CW, HID), F32),
        pltpu.VMEM_SHARED((NP, HID), F32),
        pltpu.SemaphoreType.DMA,
        pltpu.SemaphoreType.DMA,
    ]

    @functools.partial(
        pl.kernel, out_type=out_type, mesh=_mesh(), scratch_types=scratch,
        compiler_params=pltpu.CompilerParams(use_tc_tiling_on_sc=False))
    def passE(edges1d, ve_hbm, ra_hbm, den2_hbm, zacc, agg_hbm,
              dflat, dstv, div, raf, w2f, den2rows, ve, wv, sh_acc,
              semA, semB):
        c = lax.axis_index("c")
        s = lax.axis_index("s")

        @pl.when(s < 5)
        def _():
            pltpu.sync_copy(zacc, sh_acc.at[pl.ds(s * 2048, 2048)])

        plsc.subcore_barrier()

        iota = _iota16()
        cidx = [(iota * 0) + kk for kk in range(16)]
        lo, hi = _span(s, _EMC)
        for ts in range(nsrc):
            rabase = (c * nsrc + ts) * E * HG
            ebase = (ts * 2 + 1) * E

            pltpu.async_copy(edges1d.at[pl.ds(ebase + lo * _EM, _EM)],
                             dflat.at[lo & 1], semA)

            def macro(m, cr, ts=ts, rabase=rabase, ebase=ebase):
                b = m & 1
                eoff = ebase + m * _EM
                pltpu.make_async_copy(edges1d.at[pl.ds(eoff, _EM)],
                                      dflat.at[b], semA).wait()
                for i in range(CW // 16):
                    w = dflat[b, pl.ds(i * 16, 16)]
                    dstv[0, pl.ds(i * 16, 16)] = w
                    div[0, pl.ds(i * 16, 16)] = w + c * NP
                d1 = pltpu.async_copy(den2_hbm.at[div.at[0]], den2rows, semB)
                d2 = pltpu.async_copy(
                    ve_hbm.at[pl.ds(ts * E + m * _EM, _EM)], ve, semB)
                d3 = pltpu.async_copy(
                    ra_hbm.at[pl.ds(rabase + m * _EM * HG, _EM * HG)],
                    raf, semB)

                @pl.when(m + 1 < hi)
                def _():
                    pltpu.async_copy(edges1d.at[pl.ds(eoff + _EM, _EM)],
                                     dflat.at[1 - b], semA)

                d1.wait()
                d2.wait()
                d3.wait()

                def quad(q, qcr):
                    sl = pl.ds(q * 16, 16)
                    den16 = _merge_den(den2rows, q)
                    w2f[sl] = jnp.exp(-raf[sl]) / (den16 + 1e-16)
                    return qcr

                lax.fori_loop(0, _EM // 4, quad, 0)

                def quad2(q2, qcr):
                    rwin = raf[pl.ds(q2 * 16, 16)]
                    wwin = w2f[pl.ds(q2 * 16, 16)]
                    for j2 in range(4):
                        e = q2 * 4 + j2
                        for h in range(HG):
                            vv = ve[e, pl.ds(c * HW + h * DK, DK)]
                            w1s = jnp.take(rwin, cidx[j2 * 4 + h])
                            w2s = jnp.take(wwin, cidx[j2 * 4 + h])
                            wv[e, pl.ds(h * DK, DK)] = vv * w1s
                            wv[e, pl.ds(HW + h * DK, DK)] = vv * w2s
                    return qcr

                lax.fori_loop(0, CW // 4, quad2, 0)
                pltpu.sync_copy(wv, sh_acc.at[dstv.at[0]], add=True)
                return cr

            lax.fori_loop(lo, hi, macro, 0)

        plsc.subcore_barrier()

        @pl.when(s < 5)
        def _():
            pltpu.sync_copy(sh_acc.at[pl.ds(s * 2048, 2048)],
                            agg_hbm.at[pl.ds(c * NP + s * 2048, 2048)])

    return passE


_gatherA = _make_gatherA()
_passC = [_make_passC(t) for t in range(T)]
_passD = [_make_passD(t) for t in range(T)]
_passE = [_make_passE(t) for t in range(T)]

_SEL = np.zeros((2, HID, HG), dtype=np.float32)
for _h in range(H):
    _SEL[_h // HG, _h * DK:(_h + 1) * DK, _h % HG] = 1.0


def kernel(x_list, edge_index_list, Wq, bq, Wk, bk, Wv, bv,
           ln_g, ln_b, W1, b1, W2, b2):
    xf = x_list.reshape(T * N, D_IN)
    q, k, v = _proj(xf, Wq, Wk, Wv,
                    bq.reshape(1, HID), bk.reshape(1, HID), bv.reshape(1, HID))

    qtab, ktab, vtab = q, k, v
    edges1d = edge_index_list.reshape(T * 2 * E)

    qe, ke, ve = _gatherA(qtab, ktab, vtab, edges1d)
    e1 = _att(qe, ke, jnp.asarray(_SEL)).reshape(12 * E * HG)

    zden = jnp.zeros((1280, 16), F32)
    zacc = jnp.zeros((2048, HID), F32)

    ys = []
    for t_tar in range(T):
        den = _passC[t_tar](edges1d, e1, zden)
        ra, den2 = _passD[t_tar](edges1d, e1, den, zden)
        agg = _passE[t_tar](edges1d, ve, ra, den2, zacc)
        a0, a1 = agg[:N], agg[NP:NP + N]
        causal_hat = jnp.concatenate([a0[:, :HW], a1[:, :HW]], axis=1)
        spurious_hat = jnp.concatenate([a0[:, HW:], a1[:, HW:]], axis=1)
        ys.append(causal_hat + x_list[t_tar])
        ys.append(spurious_hat)

    y = jnp.stack(ys).reshape(2 * T * N, HID)
    f = _ffn(y, ln_g.reshape(1, HID), ln_b.reshape(1, HID),
             W1, b1.reshape(1, 2 * HID), W2, b2.reshape(1, HID))
    f = f.reshape(T, 2, N, HID)
    cs, ss = f[:, 0], f[:, 1]
    return cs + ss, cs, ss


# speedup vs baseline: 3.5056x; 1.0136x over previous
import functools
import math

import jax
import jax.numpy as jnp
import numpy as np
from jax import lax
from jax.experimental import pallas as pl
from jax.experimental.pallas import tpu as pltpu
from jax.experimental.pallas import tpu_sc as plsc

T, N, E = 3, 10000, 320000
D_IN, HID, H = 128, 128, 8
DK = HID // H
HG = 4
HW = HG * DK
NS = 16
NP = 10240
CW = 128
ECH = E // CW
QB = [(tt, ts) for tt in range(T) for ts in range(tt + 1)]
TS_OF = [ts for (_, ts) in QB]
F32 = jnp.float32
I32 = jnp.int32
_SQRT2 = math.sqrt(2.0)


def _qb0(tt):
    return tt * (tt + 1) // 2


_BLK = 2000


def _proj_body(x_ref, wq_ref, wk_ref, wv_ref, bq_ref, bk_ref, bv_ref,
               q_ref, k_ref, v_ref):
    x = x_ref[...]
    q_ref[...] = jnp.dot(x, wq_ref[...], preferred_element_type=F32) + bq_ref[...]
    k_ref[...] = jnp.dot(x, wk_ref[...], preferred_element_type=F32) + bk_ref[...]
    v_ref[...] = jnp.dot(x, wv_ref[...], preferred_element_type=F32) + bv_ref[...]


_proj = pl.pallas_call(
    _proj_body,
    grid=(T * N // _BLK,),
    in_specs=[pl.BlockSpec((_BLK, D_IN), lambda i: (i, 0))]
    + [pl.BlockSpec((D_IN, HID), lambda i: (0, 0))] * 3
    + [pl.BlockSpec((1, HID), lambda i: (0, 0))] * 3,
    out_specs=[pl.BlockSpec((_BLK, HID), lambda i: (i, 0))] * 3,
    out_shape=[jax.ShapeDtypeStruct((T * N, HID), F32)] * 3,
)


def _att_body(qe_ref, ke_ref, s_ref, e1_ref):
    p = qe_ref[...] * ke_ref[...]
    att = jnp.dot(p, s_ref[...].reshape(HID, HG), preferred_element_type=F32)
    e1_ref[...] = jnp.exp(att * 0.25)


_att = pl.pallas_call(
    _att_body,
    grid=(2, 6, E // _BLK),
    in_specs=[
        pl.BlockSpec((_BLK, HID), lambda c, b, i: (b * (E // _BLK) + i, 0)),
        pl.BlockSpec((_BLK, HID),
                     lambda c, b, i: (
                         (b - jnp.where(b >= 3, 3, jnp.where(b >= 1, 1, 0)))
                         * (E // _BLK) + i, 0)),
        pl.BlockSpec((1, HID, HG), lambda c, b, i: (c, 0, 0)),
    ],
    out_specs=pl.BlockSpec((_BLK, HG),
                           lambda c, b, i: ((c * 6 + b) * (E // _BLK) + i, 0)),
    out_shape=jax.ShapeDtypeStruct((12 * E, HG), F32),
)


def _ffn_body(y_ref, g_ref, b_ref, w1_ref, b1_ref, w2_ref, b2_ref, o_ref):
    y = y_ref[...]
    mu = jnp.mean(y, axis=-1, keepdims=True)
    var = jnp.mean((y - mu) ** 2, axis=-1, keepdims=True)
    hn = (y - mu) / jnp.sqrt(var + 1e-5) * g_ref[...] + b_ref[...]
    h1 = jnp.dot(hn, w1_ref[...], preferred_element_type=F32) + b1_ref[...]
    h1 = 0.5 * h1 * (1.0 + lax.erf(h1 / _SQRT2))
    h2 = jnp.dot(h1, w2_ref[...], preferred_element_type=F32) + b2_ref[...]
    o_ref[...] = y + h2


_ffn = pl.pallas_call(
    _ffn_body,
    grid=(2 * T * N // _BLK,),
    in_specs=[pl.BlockSpec((_BLK, HID), lambda i: (i, 0)),
              pl.BlockSpec((1, HID), lambda i: (0, 0)),
              pl.BlockSpec((1, HID), lambda i: (0, 0)),
              pl.BlockSpec((HID, 2 * HID), lambda i: (0, 0)),
              pl.BlockSpec((1, 2 * HID), lambda i: (0, 0)),
              pl.BlockSpec((2 * HID, HID), lambda i: (0, 0)),
              pl.BlockSpec((1, HID), lambda i: (0, 0))],
    out_specs=pl.BlockSpec((_BLK, HID), lambda i: (i, 0)),
    out_shape=jax.ShapeDtypeStruct((2 * T * N, HID), F32),
)


def _mesh():
    return plsc.VectorSubcoreMesh(core_axis_name="c", subcore_axis_name="s")


def _span(s, total):
    base, extra = total // NS, total % NS
    lo = s * base + jnp.minimum(s, extra)
    cnt = base + jnp.where(s < extra, 1, 0)
    return lo, lo + cnt


def _iota16():
    return lax.broadcasted_iota(I32, (16,), 0)


def _spread2d(dst_ref, flat_ref, off, nrow):
    for j in range(nrow):
        for i in range(CW // 16):
            w = flat_ref[pl.ds(j * CW + i * 16, 16)]
            if off is not None:
                w = w + off
            dst_ref[j, pl.ds(i * 16, 16)] = w


def _scatter_rows16(vals_flat_ref, rows_ref, nquad):
    iota = _iota16()
    perms = [(j * 4 + iota) & 15 for j in range(4)]
    lt4 = iota < 4

    def q_body(q, cr):
        win = vals_flat_ref[pl.ds(q * 16, 16)]
        for j in range(4):
            row = jnp.where(lt4, jnp.take(win, perms[j]), 0.0)
            rows_ref[q * 4 + j, :] = row
        return cr

    lax.fori_loop(0, nquad, q_body, 0)


def _merge_den(den_rows_ref, q):
    iota = _iota16()
    h_of = iota & 3
    m = None
    for j in range(4):
        w = jnp.take(den_rows_ref[4 * q + j, :], h_of)
        m = w if m is None else jnp.where((iota >> 2) == j, w, m)
    return m


_AM = 1280
_AMC = E // _AM
_AK = _AM // CW
_GM = 512
_GMC = E // _GM
_GK = _GM // CW


def _make_gatherA():
    out_type = (jax.ShapeDtypeStruct((6 * E, HID), F32),
                jax.ShapeDtypeStruct((3 * E, HID), F32),
                jax.ShapeDtypeStruct((3 * E, HID), F32))
    scratch = [
        pltpu.VMEM((2, _GM), I32),
        pltpu.VMEM((_GK, CW), I32),
        pltpu.VMEM((_GM, HID), F32),
        pltpu.SemaphoreType.DMA,
        pltpu.SemaphoreType.DMA,
        pltpu.SemaphoreType.DMA,
    ]

    @functools.partial(
        pl.kernel, out_type=out_type, mesh=_mesh(), scratch_types=scratch,
        compiler_params=pltpu.CompilerParams(use_tc_tiling_on_sc=False))
    def gatherA(qtab, ktab, vtab, edges1d, qe_hbm, ke_hbm, ve_hbm,
                dflat, iv, rows, semA, semG, semW):
        c = lax.axis_index("c")
        s = lax.axis_index("s")
        wid = s * 2 + c
        base, extra = _GMC // 32, _GMC % 32
        lo = wid * base + jnp.minimum(wid, extra)
        hi = lo + base + jnp.where(wid < extra, 1, 0)

        jobs = []
        for b, (tt, ts) in enumerate(QB):
            jobs.append((qtab, tt, 1, ts, qe_hbm, b))
        for ts in range(T):
            jobs.append((ktab, ts, 0, ts, ke_hbm, ts))
            jobs.append((vtab, ts, 0, ts, ve_hbm, ts))

        for tab, toff, rsel, ts, out, ob in jobs:
            ebase = (ts * 2 + rsel) * E
            pltpu.async_copy(edges1d.at[pl.ds(ebase + lo * _GM, _GM)],
                             dflat.at[lo & 1], semA)

            def macro(m, cr, tab=tab, toff=toff, ebase=ebase, out=out,
                      ob=ob):
                b = m & 1
                eoff = ebase + m * _GM
                pltpu.make_async_copy(edges1d.at[pl.ds(eoff, _GM)],
                                      dflat.at[b], semA).wait()

                @pl.when(m + 1 < hi)
                def _():
                    pltpu.async_copy(edges1d.at[pl.ds(eoff + _GM, _GM)],
                                     dflat.at[1 - b], semA)

                for j in range(_GK):
                    for i in range(CW // 16):
                        iv[j, pl.ds(i * 16, 16)] = (
                            dflat[b, pl.ds(j * CW + i * 16, 16)] + toff * N)

                @pl.when(m > lo)
                def _():
                    pltpu.make_async_copy(
                        rows, out.at[pl.ds(ob * E + (m - 1) * _GM, _GM)],
                        semW).wait()

                descs = []
                for j in range(_GK):
                    descs.append(pltpu.async_copy(
                        tab.at[iv.at[j]], rows.at[pl.ds(j * CW, CW)], semG))
                for d in descs:
                    d.wait()
                pltpu.async_copy(
                    rows, out.at[pl.ds(ob * E + m * _GM, _GM)], semW)
                return cr

            lax.fori_loop(lo, hi, macro, 0)
            pltpu.make_async_copy(
                rows, out.at[pl.ds(ob * E + (hi - 1) * _GM, _GM)],
                semW).wait()

    return gatherA


def _make_passC(t_tar):
    nsrc = t_tar + 1
    out_type = jax.ShapeDtypeStruct((2 * NP, 16), F32)
    scratch = [
        pltpu.VMEM((_AM,), I32),
        pltpu.VMEM((_AK, CW), I32),
        pltpu.VMEM((_AM * HG,), F32),
        pltpu.VMEM((_AM, 16), F32),
        pltpu.VMEM_SHARED((NP, 16), F32),
        pltpu.SemaphoreType.DMA,
    ]

    @functools.partial(
        pl.kernel, out_type=out_type, mesh=_mesh(), scratch_types=scratch,
        compiler_params=pltpu.CompilerParams(use_tc_tiling_on_sc=False))
    def passC(edges1d, e1_hbm, zden, den_hbm, dflat, dstv, e1f, vrows,
              sh_den, sem):
        c = lax.axis_index("c")
        s = lax.axis_index("s")

        @pl.when(s < 8)
        def _():
            pltpu.sync_copy(zden, sh_den.at[pl.ds(s * 1280, 1280)])

        plsc.subcore_barrier()

        lo, hi = _span(s, _AMC)
        for ts in range(nsrc):
            e1base = (c * 6 + _qb0(t_tar) + ts) * E * HG

            def macro(m, cr, ts=ts, e1base=e1base):
                eoff = (ts * 2 + 1) * E + m * _AM
                pltpu.sync_copy(edges1d.at[pl.ds(eoff, _AM)], dflat)
                _spread2d(dstv, dflat, None, _AK)
                pltpu.sync_copy(
                    e1_hbm.at[pl.ds(e1base + m * _AM * HG, _AM * HG)], e1f)
                _scatter_rows16(e1f, vrows, _AM // 4)
                descs = []
                for j in range(_AK):
                    descs.append(pltpu.async_copy(
                        vrows.at[pl.ds(j * CW, CW)],
                        sh_den.at[dstv.at[j]], sem, add=True))
                for d in descs:
                    d.wait()
                return cr

            lax.fori_loop(lo, hi, macro, 0)

        plsc.subcore_barrier()

        @pl.when(s < 8)
        def _():
            pltpu.sync_copy(sh_den.at[pl.ds(s * 1280, 1280)],
                            den_hbm.at[pl.ds(c * NP + s * 1280, 1280)])

    return passC


def _make_passD(t_tar):
    nsrc = t_tar + 1
    out_type = (jax.ShapeDtypeStruct((2 * nsrc * E * HG,), F32),
                jax.ShapeDtypeStruct((2 * NP, 16), F32))
    scratch = [
        pltpu.VMEM((_AM,), I32),
        pltpu.VMEM((_AK, CW), I32),
        pltpu.VMEM((_AK, CW), I32),
        pltpu.VMEM((_AM * HG,), F32),
        pltpu.VMEM((_AM * HG,), F32),
        pltpu.VMEM((_AM, 16), F32),
        pltpu.VMEM((_AM, 16), F32),
        pltpu.VMEM_SHARED((NP, 16), F32),
        pltpu.SemaphoreType.DMA,
    ]

    @functools.partial(
        pl.kernel, out_type=out_type, mesh=_mesh(), scratch_types=scratch,
        compiler_params=pltpu.CompilerParams(use_tc_tiling_on_sc=False))
    def passD(edges1d, e1_hbm, den_hbm, zden, ra_hbm, den2_hbm,
              dflat, dstv, div, e1f, e2f, denrows, vrows, sh_den2, sem):
        c = lax.axis_index("c")
        s = lax.axis_index("s")

        @pl.when(s < 8)
        def _():
            pltpu.sync_copy(zden, sh_den2.at[pl.ds(s * 1280, 1280)])

        plsc.subcore_barrier()

        lo, hi = _span(s, _AMC)
        for ts in range(nsrc):
            e1base = (c * 6 + _qb0(t_tar) + ts) * E * HG
            rabase = (c * nsrc + ts) * E * HG

            def macro(m, cr, ts=ts, e1base=e1base, rabase=rabase):
                eoff = (ts * 2 + 1) * E + m * _AM
                pltpu.sync_copy(edges1d.at[pl.ds(eoff, _AM)], dflat)
                _spread2d(dstv, dflat, None, _AK)
                _spread2d(div, dflat, c * NP, _AK)
                pltpu.sync_copy(
                    e1_hbm.at[pl.ds(e1base + m * _AM * HG, _AM * HG)], e1f)
                descs = []
                for j in range(_AK):
                    descs.append(pltpu.async_copy(
                        den_hbm.at[div.at[j]],
                        denrows.at[pl.ds(j * CW, CW)], sem))
                for d in descs:
                    d.wait()

                def quad(q, qcr):
                    sl = pl.ds(q * 16, 16)
                    den16 = _merge_den(denrows, q)
                    ra = e1f[sl] / (den16 + 1e-16)
                    e1f[sl] = ra
                    e2f[sl] = jnp.exp(-ra)
                    return qcr

                lax.fori_loop(0, _AM // 4, quad, 0)
                pltpu.sync_copy(
                    e1f, ra_hbm.at[pl.ds(rabase + m * _AM * HG, _AM * HG)])
                _scatter_rows16(e2f, vrows, _AM // 4)
                descs2 = []
                for j in range(_AK):
                    descs2.append(pltpu.async_copy(
                        vrows.at[pl.ds(j * CW, CW)],
                        sh_den2.at[dstv.at[j]], sem, add=True))
                for d in descs2:
                    d.wait()
                return cr

            lax.fori_loop(lo, hi, macro, 0)

        plsc.subcore_barrier()

        @pl.when(s < 8)
        def _():
            pltpu.sync_copy(sh_den2.at[pl.ds(s * 1280, 1280)],
                            den2_hbm.at[pl.ds(c * NP + s * 1280, 1280)])

    return passD


_EM = 128
_EMC = E // _EM
_EK = _EM // CW


def _make_passE(t_tar):
    nsrc = t_tar + 1
    out_type = jax.ShapeDtypeStruct((2 * NP, HID), F32)
    scratch = [
        pltpu.VMEM((2, _EM), I32),
        pltpu.VMEM((1, CW), I32),
        pltpu.VMEM((1, CW), I32),
        pltpu.VMEM((_EM * HG,), F32),
        pltpu.VMEM((_EM * HG,), F32),
        pltpu.VMEM((_EM, 16), F32),
        pltpu.VMEM((_EM, HID), F32),
        pltpu.VMEM((CW, HID), F32),
        pltpu.VMEM_SHARED((NP, HID), F32),
        pltpu.SemaphoreType.DMA,
        pltpu.SemaphoreType.DMA,
    ]

    @functools.partial(
        pl.kernel, out_type=out_type, mesh=_mesh(), scratch_types=scratch,
        compiler_params=pltpu.CompilerParams(use_tc_tiling_on_sc=False))
    def passE(edges1d, ve_hbm, ra_hbm, den2_hbm, zacc, agg_hbm,
              dflat, dstv, div, raf, w2f, den2rows, ve, wv, sh_acc,
              semA, semB):
        c = lax.axis_index("c")
        s = lax.axis_index("s")

        @pl.when(s < 5)
        def _():
            pltpu.sync_copy(zacc, sh_acc.at[pl.ds(s * 2048, 2048)])

        plsc.subcore_barrier()

        iota = _iota16()
        cidx = [(iota * 0) + kk for kk in range(16)]
        lo, hi = _span(s, _EMC)
        for ts in range(nsrc):
            rabase = (c * nsrc + ts) * E * HG
            ebase = (ts * 2 + 1) * E

            pltpu.async_copy(edges1d.at[pl.ds(ebase + lo * _EM, _EM)],
                             dflat.at[lo & 1], semA)

            def macro(m, cr, ts=ts, rabase=rabase, ebase=ebase):
                b = m & 1
                eoff = ebase + m * _EM
                pltpu.make_async_copy(edges1d.at[pl.ds(eoff, _EM)],
                                      dflat.at[b], semA).wait()
                for i in range(CW // 16):
                    w = dflat[b, pl.ds(i * 16, 16)]
                    dstv[0, pl.ds(i * 16, 16)] = w
                    div[0, pl.ds(i * 16, 16)] = w + c * NP
                d1 = pltpu.async_copy(den2_hbm.at[div.at[0]], den2rows, semB)
                d2 = pltpu.async_copy(
                    ve_hbm.at[pl.ds(ts * E + m * _EM, _EM)], ve, semB)
                d3 = pltpu.async_copy(
                    ra_hbm.at[pl.ds(rabase + m * _EM * HG, _EM * HG)],
                    raf, semB)

                @pl.when(m + 1 < hi)
                def _():
                    pltpu.async_copy(edges1d.at[pl.ds(eoff + _EM, _EM)],
                                     dflat.at[1 - b], semA)

                d1.wait()
                d2.wait()
                d3.wait()

                def quad(q, qcr):
                    sl = pl.ds(q * 16, 16)
                    den16 = _merge_den(den2rows, q)
                    w2f[sl] = jnp.exp(-raf[sl]) / (den16 + 1e-16)
                    return qcr

                lax.fori_loop(0, _EM // 4, quad, 0)

                def quad2(q2, qcr):
                    rwin = raf[pl.ds(q2 * 16, 16)]
                    wwin = w2f[pl.ds(q2 * 16, 16)]
                    for j2 in range(4):
                        e = q2 * 4 + j2
                        for h in range(HG):
                            vv = ve[e, pl.ds(c * HW + h * DK, DK)]
                            w1s = jnp.take(rwin, cidx[j2 * 4 + h])
                            w2s = jnp.take(wwin, cidx[j2 * 4 + h])
                            wv[e, pl.ds(h * DK, DK)] = vv * w1s
                            wv[e, pl.ds(HW + h * DK, DK)] = vv * w2s
                    return qcr

                lax.fori_loop(0, CW // 4, quad2, 0)
                pltpu.sync_copy(wv, sh_acc.at[dstv.at[0]], add=True)
                return cr

            lax.fori_loop(lo, hi, macro, 0)

        plsc.subcore_barrier()

        @pl.when(s < 5)
        def _():
            pltpu.sync_copy(sh_acc.at[pl.ds(s * 2048, 2048)],
                            agg_hbm.at[pl.ds(c * NP + s * 2048, 2048)])

    return passE


_gatherA = _make_gatherA()
_passC = [_make_passC(t) for t in range(T)]
_passD = [_make_passD(t) for t in range(T)]
_passE = [_make_passE(t) for t in range(T)]

_SEL = np.zeros((2, HID, HG), dtype=np.float32)
for _h in range(H):
    _SEL[_h // HG, _h * DK:(_h + 1) * DK, _h % HG] = 1.0


def kernel(x_list, edge_index_list, Wq, bq, Wk, bk, Wv, bv,
           ln_g, ln_b, W1, b1, W2, b2):
    xf = x_list.reshape(T * N, D_IN)
    q, k, v = _proj(xf, Wq, Wk, Wv,
                    bq.reshape(1, HID), bk.reshape(1, HID), bv.reshape(1, HID))

    qtab, ktab, vtab = q, k, v
    edges1d = edge_index_list.reshape(T * 2 * E)

    qe, ke, ve = _gatherA(qtab, ktab, vtab, edges1d)
    e1 = _att(qe, ke, jnp.asarray(_SEL)).reshape(12 * E * HG)

    zden = jnp.zeros((1280, 16), F32)
    zacc = jnp.zeros((2048, HID), F32)

    ys = []
    for t_tar in range(T):
        den = _passC[t_tar](edges1d, e1, zden)
        ra, den2 = _passD[t_tar](edges1d, e1, den, zden)
        agg = _passE[t_tar](edges1d, ve, ra, den2, zacc)
        a0, a1 = agg[:N], agg[NP:NP + N]
        causal_hat = jnp.concatenate([a0[:, :HW], a1[:, :HW]], axis=1)
        spurious_hat = jnp.concatenate([a0[:, HW:], a1[:, HW:]], axis=1)
        ys.append(causal_hat + x_list[t_tar])
        ys.append(spurious_hat)

    y = jnp.stack(ys).reshape(2 * T * N, HID)
    f = _ffn(y, ln_g.reshape(1, HID), ln_b.reshape(1, HID),
             W1, b1.reshape(1, 2 * HID), W2, b2.reshape(1, HID))
    f = f.reshape(T, 2, N, HID)
    cs, ss = f[:, 0], f[:, 1]
    return cs + ss, cs, ss


# speedup vs baseline: 3.5945x; 1.0254x over previous
import functools
import math

import jax
import jax.numpy as jnp
import numpy as np
from jax import lax
from jax.experimental import pallas as pl
from jax.experimental.pallas import tpu as pltpu
from jax.experimental.pallas import tpu_sc as plsc

T, N, E = 3, 10000, 320000
D_IN, HID, H = 128, 128, 8
DK = HID // H
HG = 4
HW = HG * DK
NS = 16
NP = 10240
CW = 128
ECH = E // CW
QB = [(tt, ts) for tt in range(T) for ts in range(tt + 1)]
TS_OF = [ts for (_, ts) in QB]
F32 = jnp.float32
I32 = jnp.int32
_SQRT2 = math.sqrt(2.0)


def _qb0(tt):
    return tt * (tt + 1) // 2


_BLK = 2000


def _proj_body(x_ref, wq_ref, wk_ref, wv_ref, bq_ref, bk_ref, bv_ref,
               q_ref, k_ref, v_ref):
    x = x_ref[...]
    q_ref[...] = jnp.dot(x, wq_ref[...], preferred_element_type=F32) + bq_ref[...]
    k_ref[...] = jnp.dot(x, wk_ref[...], preferred_element_type=F32) + bk_ref[...]
    v_ref[...] = jnp.dot(x, wv_ref[...], preferred_element_type=F32) + bv_ref[...]


_proj = pl.pallas_call(
    _proj_body,
    grid=(T * N // _BLK,),
    in_specs=[pl.BlockSpec((_BLK, D_IN), lambda i: (i, 0))]
    + [pl.BlockSpec((D_IN, HID), lambda i: (0, 0))] * 3
    + [pl.BlockSpec((1, HID), lambda i: (0, 0))] * 3,
    out_specs=[pl.BlockSpec((_BLK, HID), lambda i: (i, 0))] * 3,
    out_shape=[jax.ShapeDtypeStruct((T * N, HID), F32)] * 3,
)


def _att_body(qe_ref, ke_ref, s_ref, e1_ref):
    p = qe_ref[...] * ke_ref[...]
    att = jnp.dot(p, s_ref[...].reshape(HID, HG), preferred_element_type=F32)
    e1_ref[...] = jnp.exp(att * 0.25)


_att = pl.pallas_call(
    _att_body,
    grid=(2, 6, E // _BLK),
    in_specs=[
        pl.BlockSpec((_BLK, HID), lambda c, b, i: (b * (E // _BLK) + i, 0)),
        pl.BlockSpec((_BLK, HID),
                     lambda c, b, i: (
                         (b - jnp.where(b >= 3, 3, jnp.where(b >= 1, 1, 0)))
                         * (E // _BLK) + i, 0)),
        pl.BlockSpec((1, HID, HG), lambda c, b, i: (c, 0, 0)),
    ],
    out_specs=pl.BlockSpec((_BLK, HG),
                           lambda c, b, i: ((c * 6 + b) * (E // _BLK) + i, 0)),
    out_shape=jax.ShapeDtypeStruct((12 * E, HG), F32),
)


def _ffn_body(y_ref, g_ref, b_ref, w1_ref, b1_ref, w2_ref, b2_ref, o_ref):
    y = y_ref[...]
    mu = jnp.mean(y, axis=-1, keepdims=True)
    var = jnp.mean((y - mu) ** 2, axis=-1, keepdims=True)
    hn = (y - mu) / jnp.sqrt(var + 1e-5) * g_ref[...] + b_ref[...]
    h1 = jnp.dot(hn, w1_ref[...], preferred_element_type=F32) + b1_ref[...]
    h1 = 0.5 * h1 * (1.0 + lax.erf(h1 / _SQRT2))
    h2 = jnp.dot(h1, w2_ref[...], preferred_element_type=F32) + b2_ref[...]
    o_ref[...] = y + h2


_ffn = pl.pallas_call(
    _ffn_body,
    grid=(2 * T * N // _BLK,),
    in_specs=[pl.BlockSpec((_BLK, HID), lambda i: (i, 0)),
              pl.BlockSpec((1, HID), lambda i: (0, 0)),
              pl.BlockSpec((1, HID), lambda i: (0, 0)),
              pl.BlockSpec((HID, 2 * HID), lambda i: (0, 0)),
              pl.BlockSpec((1, 2 * HID), lambda i: (0, 0)),
              pl.BlockSpec((2 * HID, HID), lambda i: (0, 0)),
              pl.BlockSpec((1, HID), lambda i: (0, 0))],
    out_specs=pl.BlockSpec((_BLK, HID), lambda i: (i, 0)),
    out_shape=jax.ShapeDtypeStruct((2 * T * N, HID), F32),
)


def _mesh():
    return plsc.VectorSubcoreMesh(core_axis_name="c", subcore_axis_name="s")


def _span(s, total):
    base, extra = total // NS, total % NS
    lo = s * base + jnp.minimum(s, extra)
    cnt = base + jnp.where(s < extra, 1, 0)
    return lo, lo + cnt


def _iota16():
    return lax.broadcasted_iota(I32, (16,), 0)


def _spread2d(dst_ref, flat_ref, off, nrow):
    for j in range(nrow):
        for i in range(CW // 16):
            w = flat_ref[pl.ds(j * CW + i * 16, 16)]
            if off is not None:
                w = w + off
            dst_ref[j, pl.ds(i * 16, 16)] = w


def _scatter_rows16(vals_flat_ref, rows_ref, nquad):
    iota = _iota16()
    perms = [(j * 4 + iota) & 15 for j in range(4)]
    lt4 = iota < 4

    def q_body(q, cr):
        win = vals_flat_ref[pl.ds(q * 16, 16)]
        for j in range(4):
            row = jnp.where(lt4, jnp.take(win, perms[j]), 0.0)
            rows_ref[q * 4 + j, :] = row
        return cr

    lax.fori_loop(0, nquad, q_body, 0)


def _merge_den(den_rows_ref, q):
    iota = _iota16()
    h_of = iota & 3
    m = None
    for j in range(4):
        w = jnp.take(den_rows_ref[4 * q + j, :], h_of)
        m = w if m is None else jnp.where((iota >> 2) == j, w, m)
    return m


_AM = 1280
_AMC = E // _AM
_AK = _AM // CW
_GM = 512
_GMC = E // _GM
_GK = _GM // CW


def _make_gatherA():
    out_type = (jax.ShapeDtypeStruct((6 * E, HID), F32),
                jax.ShapeDtypeStruct((3 * E, HID), F32),
                jax.ShapeDtypeStruct((3 * E, HID), F32))
    scratch = [
        pltpu.VMEM((2, _GM), I32),
        pltpu.VMEM((_GK, CW), I32),
        pltpu.VMEM((_GM, HID), F32),
        pltpu.SemaphoreType.DMA,
        pltpu.SemaphoreType.DMA,
        pltpu.SemaphoreType.DMA,
    ]

    @functools.partial(
        pl.kernel, out_type=out_type, mesh=_mesh(), scratch_types=scratch,
        compiler_params=pltpu.CompilerParams(use_tc_tiling_on_sc=False))
    def gatherA(qtab, ktab, vtab, edges1d, qe_hbm, ke_hbm, ve_hbm,
                dflat, iv, rows, semA, semG, semW):
        c = lax.axis_index("c")
        s = lax.axis_index("s")
        wid = s * 2 + c
        base, extra = _GMC // 32, _GMC % 32
        lo = wid * base + jnp.minimum(wid, extra)
        hi = lo + base + jnp.where(wid < extra, 1, 0)

        jobs = []
        for b, (tt, ts) in enumerate(QB):
            jobs.append((qtab, tt, 1, ts, qe_hbm, b))
        for ts in range(T):
            jobs.append((ktab, ts, 0, ts, ke_hbm, ts))
            jobs.append((vtab, ts, 0, ts, ve_hbm, ts))

        for tab, toff, rsel, ts, out, ob in jobs:
            ebase = (ts * 2 + rsel) * E
            pltpu.async_copy(edges1d.at[pl.ds(ebase + lo * _GM, _GM)],
                             dflat.at[lo & 1], semA)

            def macro(m, cr, tab=tab, toff=toff, ebase=ebase, out=out,
                      ob=ob):
                b = m & 1
                eoff = ebase + m * _GM
                pltpu.make_async_copy(edges1d.at[pl.ds(eoff, _GM)],
                                      dflat.at[b], semA).wait()

                @pl.when(m + 1 < hi)
                def _():
                    pltpu.async_copy(edges1d.at[pl.ds(eoff + _GM, _GM)],
                                     dflat.at[1 - b], semA)

                for j in range(_GK):
                    for i in range(CW // 16):
                        iv[j, pl.ds(i * 16, 16)] = (
                            dflat[b, pl.ds(j * CW + i * 16, 16)] + toff * N)

                @pl.when(m > lo)
                def _():
                    pltpu.make_async_copy(
                        rows, out.at[pl.ds(ob * E + (m - 1) * _GM, _GM)],
                        semW).wait()

                descs = []
                for j in range(_GK):
                    descs.append(pltpu.async_copy(
                        tab.at[iv.at[j]], rows.at[pl.ds(j * CW, CW)], semG))
                for d in descs:
                    d.wait()
                pltpu.async_copy(
                    rows, out.at[pl.ds(ob * E + m * _GM, _GM)], semW)
                return cr

            lax.fori_loop(lo, hi, macro, 0)
            pltpu.make_async_copy(
                rows, out.at[pl.ds(ob * E + (hi - 1) * _GM, _GM)],
                semW).wait()

    return gatherA


def _make_passCD(t_tar):
    nsrc = t_tar + 1
    out_type = (jax.ShapeDtypeStruct((2 * nsrc * E * HG,), F32),
                jax.ShapeDtypeStruct((2 * NP, 16), F32))
    scratch = [
        pltpu.VMEM((2, _AM), I32),
        pltpu.VMEM((_AK, CW), I32),
        pltpu.VMEM((2, _AM * HG), F32),
        pltpu.VMEM((_AM * HG,), F32),
        pltpu.VMEM((_AM, 16), F32),
        pltpu.VMEM((_AM, 16), F32),
        pltpu.VMEM_SHARED((NP, 16), F32),
        pltpu.VMEM_SHARED((NP, 16), F32),
        pltpu.SemaphoreType.DMA,
        pltpu.SemaphoreType.DMA,
    ]

    @functools.partial(
        pl.kernel, out_type=out_type, mesh=_mesh(), scratch_types=scratch,
        compiler_params=pltpu.CompilerParams(use_tc_tiling_on_sc=False))
    def passCD(edges1d, e1_hbm, zden, ra_hbm, den2_hbm,
               dflat, dstv, e1f, e2f, denrows, vrows, sh_den, sh_den2,
               semA, semB):
        c = lax.axis_index("c")
        s = lax.axis_index("s")

        @pl.when(s < 8)
        def _():
            pltpu.sync_copy(zden, sh_den.at[pl.ds(s * 1280, 1280)])
            pltpu.sync_copy(zden, sh_den2.at[pl.ds(s * 1280, 1280)])

        plsc.subcore_barrier()

        lo, hi = _span(s, _AMC)

        def prefetch(ts, m, b):
            eoff = (ts * 2 + 1) * E + m * _AM
            e1off = (c * 6 + _qb0(t_tar) + ts) * E * HG + m * _AM * HG
            pltpu.async_copy(edges1d.at[pl.ds(eoff, _AM)], dflat.at[b], semA)
            pltpu.async_copy(e1_hbm.at[pl.ds(e1off, _AM * HG)],
                             e1f.at[b], semA)

        def wait_prefetch(ts, m, b):
            eoff = (ts * 2 + 1) * E + m * _AM
            e1off = (c * 6 + _qb0(t_tar) + ts) * E * HG + m * _AM * HG
            pltpu.make_async_copy(edges1d.at[pl.ds(eoff, _AM)],
                                  dflat.at[b], semA).wait()
            pltpu.make_async_copy(e1_hbm.at[pl.ds(e1off, _AM * HG)],
                                  e1f.at[b], semA).wait()

        for ts in range(nsrc):
            prefetch(ts, lo, lo & 1)

            def macro(m, cr, ts=ts):
                b = m & 1
                wait_prefetch(ts, m, b)

                @pl.when(m + 1 < hi)
                def _():
                    prefetch(ts, m + 1, 1 - b)

                for j in range(_AK):
                    for i in range(CW // 16):
                        dstv[j, pl.ds(i * 16, 16)] = dflat[
                            b, pl.ds(j * CW + i * 16, 16)]
                _scatter_rows16(e1f.at[b], vrows, _AM // 4)
                descs = []
                for j in range(_AK):
                    descs.append(pltpu.async_copy(
                        vrows.at[pl.ds(j * CW, CW)],
                        sh_den.at[dstv.at[j]], semB, add=True))
                for d in descs:
                    d.wait()
                return cr

            lax.fori_loop(lo, hi, macro, 0)

        plsc.subcore_barrier()

        for ts in range(nsrc):
            rabase = (c * nsrc + ts) * E * HG
            prefetch(ts, lo, lo & 1)

            def macro2(m, cr, ts=ts, rabase=rabase):
                b = m & 1
                wait_prefetch(ts, m, b)

                @pl.when(m + 1 < hi)
                def _():
                    prefetch(ts, m + 1, 1 - b)

                for j in range(_AK):
                    for i in range(CW // 16):
                        dstv[j, pl.ds(i * 16, 16)] = dflat[
                            b, pl.ds(j * CW + i * 16, 16)]
                descs = []
                for j in range(_AK):
                    descs.append(pltpu.async_copy(
                        sh_den.at[dstv.at[j]],
                        denrows.at[pl.ds(j * CW, CW)], semB))
                for d in descs:
                    d.wait()

                def quad(q, qcr):
                    sl = pl.ds(q * 16, 16)
                    den16 = _merge_den(denrows, q)
                    ra = e1f[b, sl] / (den16 + 1e-16)
                    e1f[b, sl] = ra
                    e2f[sl] = jnp.exp(-ra)
                    return qcr

                lax.fori_loop(0, _AM // 4, quad, 0)
                pltpu.sync_copy(
                    e1f.at[b],
                    ra_hbm.at[pl.ds(rabase + m * _AM * HG, _AM * HG)])
                _scatter_rows16(e2f, vrows, _AM // 4)
                descs2 = []
                for j in range(_AK):
                    descs2.append(pltpu.async_copy(
                        vrows.at[pl.ds(j * CW, CW)],
                        sh_den2.at[dstv.at[j]], semB, add=True))
                for d in descs2:
                    d.wait()
                return cr

            lax.fori_loop(lo, hi, macro2, 0)

        plsc.subcore_barrier()

        @pl.when(s < 8)
        def _():
            pltpu.sync_copy(sh_den2.at[pl.ds(s * 1280, 1280)],
                            den2_hbm.at[pl.ds(c * NP + s * 1280, 1280)])

    return passCD


_EM = 128
_EMC = E // _EM
_EK = _EM // CW


def _make_passE(t_tar):
    nsrc = t_tar + 1
    out_type = jax.ShapeDtypeStruct((2 * NP, HID), F32)
    scratch = [
        pltpu.VMEM((2, _EM), I32),
        pltpu.VMEM((1, CW), I32),
        pltpu.VMEM((1, CW), I32),
        pltpu.VMEM((_EM * HG,), F32),
        pltpu.VMEM((_EM * HG,), F32),
        pltpu.VMEM((_EM, 16), F32),
        pltpu.VMEM((_EM, HID), F32),
        pltpu.VMEM((CW, HID), F32),
        pltpu.VMEM_SHARED((NP, HID), F32),
        pltpu.SemaphoreType.DMA,
        pltpu.SemaphoreType.DMA,
    ]

    @functools.partial(
        pl.kernel, out_type=out_type, mesh=_mesh(), scratch_types=scratch,
        compiler_params=pltpu.CompilerParams(use_tc_tiling_on_sc=False))
    def passE(edges1d, ve_hbm, ra_hbm, den2_hbm, zacc, agg_hbm,
              dflat, dstv, div, raf, w2f, den2rows, ve, wv, sh_acc,
              semA, semB):
        c = lax.axis_index("c")
        s = lax.axis_index("s")

        @pl.when(s < 5)
        def _():
            pltpu.sync_copy(zacc, sh_acc.at[pl.ds(s * 2048, 2048)])

        plsc.subcore_barrier()

        iota = _iota16()
        cidx = [(iota * 0) + kk for kk in range(16)]
        lo, hi = _span(s, _EMC)
        for ts in range(nsrc):
            rabase = (c * nsrc + ts) * E * HG
            ebase = (ts * 2 + 1) * E

            pltpu.async_copy(edges1d.at[pl.ds(ebase + lo * _EM, _EM)],
                             dflat.at[lo & 1], semA)

            def macro(m, cr, ts=ts, rabase=rabase, ebase=ebase):
                b = m & 1
                eoff = ebase + m * _EM
                pltpu.make_async_copy(edges1d.at[pl.ds(eoff, _EM)],
                                      dflat.at[b], semA).wait()
                for i in range(CW // 16):
                    w = dflat[b, pl.ds(i * 16, 16)]
                    dstv[0, pl.ds(i * 16, 16)] = w
                    div[0, pl.ds(i * 16, 16)] = w + c * NP
                d1 = pltpu.async_copy(den2_hbm.at[div.at[0]], den2rows, semB)
                d2 = pltpu.async_copy(
                    ve_hbm.at[pl.ds(ts * E + m * _EM, _EM)], ve, semB)
                d3 = pltpu.async_copy(
                    ra_hbm.at[pl.ds(rabase + m * _EM * HG, _EM * HG)],
                    raf, semB)

                @pl.when(m + 1 < hi)
                def _():
                    pltpu.async_copy(edges1d.at[pl.ds(eoff + _EM, _EM)],
                                     dflat.at[1 - b], semA)

                d1.wait()
                d2.wait()
                d3.wait()

                def quad(q, qcr):
                    sl = pl.ds(q * 16, 16)
                    den16 = _merge_den(den2rows, q)
                    w2f[sl] = jnp.exp(-raf[sl]) / (den16 + 1e-16)
                    return qcr

                lax.fori_loop(0, _EM // 4, quad, 0)

                def quad2(q2, qcr):
                    rwin = raf[pl.ds(q2 * 16, 16)]
                    wwin = w2f[pl.ds(q2 * 16, 16)]
                    for j2 in range(4):
                        e = q2 * 4 + j2
                        for h in range(HG):
                            vv = ve[e, pl.ds(c * HW + h * DK, DK)]
                            w1s = jnp.take(rwin, cidx[j2 * 4 + h])
                            w2s = jnp.take(wwin, cidx[j2 * 4 + h])
                            wv[e, pl.ds(h * DK, DK)] = vv * w1s
                            wv[e, pl.ds(HW + h * DK, DK)] = vv * w2s
                    return qcr

                lax.fori_loop(0, CW // 4, quad2, 0)
                pltpu.sync_copy(wv, sh_acc.at[dstv.at[0]], add=True)
                return cr

            lax.fori_loop(lo, hi, macro, 0)

        plsc.subcore_barrier()

        @pl.when(s < 5)
        def _():
            pltpu.sync_copy(sh_acc.at[pl.ds(s * 2048, 2048)],
                            agg_hbm.at[pl.ds(c * NP + s * 2048, 2048)])

    return passE


_gatherA = _make_gatherA()
_passCD = [_make_passCD(t) for t in range(T)]
_passE = [_make_passE(t) for t in range(T)]

_SEL = np.zeros((2, HID, HG), dtype=np.float32)
for _h in range(H):
    _SEL[_h // HG, _h * DK:(_h + 1) * DK, _h % HG] = 1.0


def kernel(x_list, edge_index_list, Wq, bq, Wk, bk, Wv, bv,
           ln_g, ln_b, W1, b1, W2, b2):
    xf = x_list.reshape(T * N, D_IN)
    q, k, v = _proj(xf, Wq, Wk, Wv,
                    bq.reshape(1, HID), bk.reshape(1, HID), bv.reshape(1, HID))

    qtab, ktab, vtab = q, k, v
    edges1d = edge_index_list.reshape(T * 2 * E)

    qe, ke, ve = _gatherA(qtab, ktab, vtab, edges1d)
    e1 = _att(qe, ke, jnp.asarray(_SEL)).reshape(12 * E * HG)

    zden = jnp.zeros((1280, 16), F32)
    zacc = jnp.zeros((2048, HID), F32)

    ys = []
    for t_tar in range(T):
        ra, den2 = _passCD[t_tar](edges1d, e1, zden)
        agg = _passE[t_tar](edges1d, ve, ra, den2, zacc)
        a0, a1 = agg[:N], agg[NP:NP + N]
        causal_hat = jnp.concatenate([a0[:, :HW], a1[:, :HW]], axis=1)
        spurious_hat = jnp.concatenate([a0[:, HW:], a1[:, HW:]], axis=1)
        ys.append(causal_hat + x_list[t_tar])
        ys.append(spurious_hat)

    y = jnp.stack(ys).reshape(2 * T * N, HID)
    f = _ffn(y, ln_g.reshape(1, HID), ln_b.reshape(1, HID),
             W1, b1.reshape(1, 2 * HID), W2, b2.reshape(1, HID))
    f = f.reshape(T, 2, N, HID)
    cs, ss = f[:, 0], f[:, 1]
    return cs + ss, cs, ss


# speedup vs baseline: 3.7715x; 1.0492x over previous
import functools
import math

import jax
import jax.numpy as jnp
import numpy as np
from jax import lax
from jax.experimental import pallas as pl
from jax.experimental.pallas import tpu as pltpu
from jax.experimental.pallas import tpu_sc as plsc

T, N, E = 3, 10000, 320000
D_IN, HID, H = 128, 128, 8
DK = HID // H
HG = 4
HW = HG * DK
NS = 16
NP = 10240
CW = 128
ECH = E // CW
QB = [(tt, ts) for tt in range(T) for ts in range(tt + 1)]
TS_OF = [ts for (_, ts) in QB]
F32 = jnp.float32
I32 = jnp.int32
_SQRT2 = math.sqrt(2.0)


def _qb0(tt):
    return tt * (tt + 1) // 2


_BLK = 2000


def _proj_body(x_ref, wq_ref, wk_ref, wv_ref, bq_ref, bk_ref, bv_ref,
               q_ref, k_ref, v_ref):
    x = x_ref[...]
    q_ref[...] = jnp.dot(x, wq_ref[...], preferred_element_type=F32) + bq_ref[...]
    k_ref[...] = jnp.dot(x, wk_ref[...], preferred_element_type=F32) + bk_ref[...]
    v_ref[...] = jnp.dot(x, wv_ref[...], preferred_element_type=F32) + bv_ref[...]


_proj = pl.pallas_call(
    _proj_body,
    grid=(T * N // _BLK,),
    in_specs=[pl.BlockSpec((_BLK, D_IN), lambda i: (i, 0))]
    + [pl.BlockSpec((D_IN, HID), lambda i: (0, 0))] * 3
    + [pl.BlockSpec((1, HID), lambda i: (0, 0))] * 3,
    out_specs=[pl.BlockSpec((_BLK, HID), lambda i: (i, 0))] * 3,
    out_shape=[jax.ShapeDtypeStruct((T * N, HID), F32)] * 3,
)


def _att_body(qe_ref, ke_ref, s_ref, e1_ref):
    p = qe_ref[...] * ke_ref[...]
    att = jnp.dot(p, s_ref[...].reshape(HID, HG), preferred_element_type=F32)
    e1_ref[...] = jnp.exp(att * 0.25)


_att = pl.pallas_call(
    _att_body,
    grid=(2, 6, E // _BLK),
    in_specs=[
        pl.BlockSpec((_BLK, HID), lambda c, b, i: (b * (E // _BLK) + i, 0)),
        pl.BlockSpec((_BLK, HID),
                     lambda c, b, i: (
                         (b - jnp.where(b >= 3, 3, jnp.where(b >= 1, 1, 0)))
                         * (E // _BLK) + i, 0)),
        pl.BlockSpec((1, HID, HG), lambda c, b, i: (c, 0, 0)),
    ],
    out_specs=pl.BlockSpec((_BLK, HG),
                           lambda c, b, i: ((c * 6 + b) * (E // _BLK) + i, 0)),
    out_shape=jax.ShapeDtypeStruct((12 * E, HG), F32),
)


def _ffn_body(y_ref, g_ref, b_ref, w1_ref, b1_ref, w2_ref, b2_ref, o_ref):
    y = y_ref[...]
    mu = jnp.mean(y, axis=-1, keepdims=True)
    var = jnp.mean((y - mu) ** 2, axis=-1, keepdims=True)
    hn = (y - mu) / jnp.sqrt(var + 1e-5) * g_ref[...] + b_ref[...]
    h1 = jnp.dot(hn, w1_ref[...], preferred_element_type=F32) + b1_ref[...]
    h1 = 0.5 * h1 * (1.0 + lax.erf(h1 / _SQRT2))
    h2 = jnp.dot(h1, w2_ref[...], preferred_element_type=F32) + b2_ref[...]
    o_ref[...] = y + h2


_ffn = pl.pallas_call(
    _ffn_body,
    grid=(2 * T * N // _BLK,),
    in_specs=[pl.BlockSpec((_BLK, HID), lambda i: (i, 0)),
              pl.BlockSpec((1, HID), lambda i: (0, 0)),
              pl.BlockSpec((1, HID), lambda i: (0, 0)),
              pl.BlockSpec((HID, 2 * HID), lambda i: (0, 0)),
              pl.BlockSpec((1, 2 * HID), lambda i: (0, 0)),
              pl.BlockSpec((2 * HID, HID), lambda i: (0, 0)),
              pl.BlockSpec((1, HID), lambda i: (0, 0))],
    out_specs=pl.BlockSpec((_BLK, HID), lambda i: (i, 0)),
    out_shape=jax.ShapeDtypeStruct((2 * T * N, HID), F32),
)


def _mesh():
    return plsc.VectorSubcoreMesh(core_axis_name="c", subcore_axis_name="s")


def _span(s, total):
    base, extra = total // NS, total % NS
    lo = s * base + jnp.minimum(s, extra)
    cnt = base + jnp.where(s < extra, 1, 0)
    return lo, lo + cnt


def _iota16():
    return lax.broadcasted_iota(I32, (16,), 0)


def _spread2d(dst_ref, flat_ref, off, nrow):
    for j in range(nrow):
        for i in range(CW // 16):
            w = flat_ref[pl.ds(j * CW + i * 16, 16)]
            if off is not None:
                w = w + off
            dst_ref[j, pl.ds(i * 16, 16)] = w


def _scatter_rows16(vals_flat_ref, rows_ref, nquad):
    iota = _iota16()
    perms = [(j * 4 + iota) & 15 for j in range(4)]
    lt4 = iota < 4

    def q_body(q, cr):
        win = vals_flat_ref[pl.ds(q * 16, 16)]
        for j in range(4):
            row = jnp.where(lt4, jnp.take(win, perms[j]), 0.0)
            rows_ref[q * 4 + j, :] = row
        return cr

    lax.fori_loop(0, nquad, q_body, 0)


def _merge_den(den_rows_ref, q):
    iota = _iota16()
    h_of = iota & 3
    m = None
    for j in range(4):
        w = jnp.take(den_rows_ref[4 * q + j, :], h_of)
        m = w if m is None else jnp.where((iota >> 2) == j, w, m)
    return m


_AM = 1280
_AMC = E // _AM
_AK = _AM // CW
_GM = 512
_GMC = E // _GM
_GK = _GM // CW


def _make_gatherA():
    out_type = (jax.ShapeDtypeStruct((6 * E, HID), F32),
                jax.ShapeDtypeStruct((3 * E, HID), F32),
                jax.ShapeDtypeStruct((3 * E, HID), F32))
    scratch = [
        pltpu.VMEM((2, _GM), I32),
        pltpu.VMEM((_GK, CW), I32),
        pltpu.VMEM((_GM, HID), F32),
        pltpu.SemaphoreType.DMA,
        pltpu.SemaphoreType.DMA,
        pltpu.SemaphoreType.DMA,
    ]

    @functools.partial(
        pl.kernel, out_type=out_type, mesh=_mesh(), scratch_types=scratch,
        compiler_params=pltpu.CompilerParams(use_tc_tiling_on_sc=False))
    def gatherA(qtab, ktab, vtab, edges1d, qe_hbm, ke_hbm, ve_hbm,
                dflat, iv, rows, semA, semG, semW):
        c = lax.axis_index("c")
        s = lax.axis_index("s")
        wid = s * 2 + c
        base, extra = _GMC // 32, _GMC % 32
        lo = wid * base + jnp.minimum(wid, extra)
        hi = lo + base + jnp.where(wid < extra, 1, 0)

        jobs = []
        for b, (tt, ts) in enumerate(QB):
            jobs.append((qtab, tt, 1, ts, qe_hbm, b))
        for ts in range(T):
            jobs.append((ktab, ts, 0, ts, ke_hbm, ts))
            jobs.append((vtab, ts, 0, ts, ve_hbm, ts))

        for tab, toff, rsel, ts, out, ob in jobs:
            ebase = (ts * 2 + rsel) * E
            pltpu.async_copy(edges1d.at[pl.ds(ebase + lo * _GM, _GM)],
                             dflat.at[lo & 1], semA)

            def macro(m, cr, tab=tab, toff=toff, ebase=ebase, out=out,
                      ob=ob):
                b = m & 1
                eoff = ebase + m * _GM
                pltpu.make_async_copy(edges1d.at[pl.ds(eoff, _GM)],
                                      dflat.at[b], semA).wait()

                @pl.when(m + 1 < hi)
                def _():
                    pltpu.async_copy(edges1d.at[pl.ds(eoff + _GM, _GM)],
                                     dflat.at[1 - b], semA)

                for j in range(_GK):
                    for i in range(CW // 16):
                        iv[j, pl.ds(i * 16, 16)] = (
                            dflat[b, pl.ds(j * CW + i * 16, 16)] + toff * N)

                @pl.when(m > lo)
                def _():
                    pltpu.make_async_copy(
                        rows, out.at[pl.ds(ob * E + (m - 1) * _GM, _GM)],
                        semW).wait()

                descs = []
                for j in range(_GK):
                    descs.append(pltpu.async_copy(
                        tab.at[iv.at[j]], rows.at[pl.ds(j * CW, CW)], semG))
                for d in descs:
                    d.wait()
                pltpu.async_copy(
                    rows, out.at[pl.ds(ob * E + m * _GM, _GM)], semW)
                return cr

            lax.fori_loop(lo, hi, macro, 0)
            pltpu.make_async_copy(
                rows, out.at[pl.ds(ob * E + (hi - 1) * _GM, _GM)],
                semW).wait()

    return gatherA


def _make_passCD(t_tar):
    nsrc = t_tar + 1
    out_type = (jax.ShapeDtypeStruct((2 * nsrc * E * HG,), F32),
                jax.ShapeDtypeStruct((2 * NP, 16), F32))
    scratch = [
        pltpu.VMEM((2, _AM), I32),
        pltpu.VMEM((_AK, CW), I32),
        pltpu.VMEM((2, _AM * HG), F32),
        pltpu.VMEM((_AM * HG,), F32),
        pltpu.VMEM((_AM, 16), F32),
        pltpu.VMEM((_AM, 16), F32),
        pltpu.VMEM_SHARED((NP, 16), F32),
        pltpu.VMEM_SHARED((NP, 16), F32),
        pltpu.SemaphoreType.DMA,
        pltpu.SemaphoreType.DMA,
    ]

    @functools.partial(
        pl.kernel, out_type=out_type, mesh=_mesh(), scratch_types=scratch,
        compiler_params=pltpu.CompilerParams(use_tc_tiling_on_sc=False))
    def passCD(edges1d, e1_hbm, zden, ra_hbm, den2_hbm,
               dflat, dstv, e1f, e2f, denrows, vrows, sh_den, sh_den2,
               semA, semB):
        c = lax.axis_index("c")
        s = lax.axis_index("s")

        @pl.when(s < 8)
        def _():
            pltpu.sync_copy(zden, sh_den.at[pl.ds(s * 1280, 1280)])
            pltpu.sync_copy(zden, sh_den2.at[pl.ds(s * 1280, 1280)])

        plsc.subcore_barrier()

        lo, hi = _span(s, _AMC)

        def prefetch(ts, m, b):
            eoff = (ts * 2 + 1) * E + m * _AM
            e1off = (c * 6 + _qb0(t_tar) + ts) * E * HG + m * _AM * HG
            pltpu.async_copy(edges1d.at[pl.ds(eoff, _AM)], dflat.at[b], semA)
            pltpu.async_copy(e1_hbm.at[pl.ds(e1off, _AM * HG)],
                             e1f.at[b], semA)

        def wait_prefetch(ts, m, b):
            eoff = (ts * 2 + 1) * E + m * _AM
            e1off = (c * 6 + _qb0(t_tar) + ts) * E * HG + m * _AM * HG
            pltpu.make_async_copy(edges1d.at[pl.ds(eoff, _AM)],
                                  dflat.at[b], semA).wait()
            pltpu.make_async_copy(e1_hbm.at[pl.ds(e1off, _AM * HG)],
                                  e1f.at[b], semA).wait()

        for ts in range(nsrc):
            prefetch(ts, lo, lo & 1)

            def macro(m, cr, ts=ts):
                b = m & 1
                wait_prefetch(ts, m, b)

                @pl.when(m + 1 < hi)
                def _():
                    prefetch(ts, m + 1, 1 - b)

                for j in range(_AK):
                    for i in range(CW // 16):
                        dstv[j, pl.ds(i * 16, 16)] = dflat[
                            b, pl.ds(j * CW + i * 16, 16)]
                _scatter_rows16(e1f.at[b], vrows, _AM // 4)
                descs = []
                for j in range(_AK):
                    descs.append(pltpu.async_copy(
                        vrows.at[pl.ds(j * CW, CW)],
                        sh_den.at[dstv.at[j]], semB, add=True))
                for d in descs:
                    d.wait()
                return cr

            lax.fori_loop(lo, hi, macro, 0)

        plsc.subcore_barrier()

        for ts in range(nsrc):
            rabase = (c * nsrc + ts) * E * HG
            prefetch(ts, lo, lo & 1)

            def macro2(m, cr, ts=ts, rabase=rabase):
                b = m & 1
                wait_prefetch(ts, m, b)

                @pl.when(m + 1 < hi)
                def _():
                    prefetch(ts, m + 1, 1 - b)

                for j in range(_AK):
                    for i in range(CW // 16):
                        dstv[j, pl.ds(i * 16, 16)] = dflat[
                            b, pl.ds(j * CW + i * 16, 16)]
                descs = []
                for j in range(_AK):
                    descs.append(pltpu.async_copy(
                        sh_den.at[dstv.at[j]],
                        denrows.at[pl.ds(j * CW, CW)], semB))
                for d in descs:
                    d.wait()

                def quad(q, qcr):
                    sl = pl.ds(q * 16, 16)
                    den16 = _merge_den(denrows, q)
                    ra = e1f[b, sl] / (den16 + 1e-16)
                    e1f[b, sl] = ra
                    e2f[sl] = jnp.exp(-ra)
                    return qcr

                lax.fori_loop(0, _AM // 4, quad, 0)
                pltpu.sync_copy(
                    e1f.at[b],
                    ra_hbm.at[pl.ds(rabase + m * _AM * HG, _AM * HG)])
                _scatter_rows16(e2f, vrows, _AM // 4)
                descs2 = []
                for j in range(_AK):
                    descs2.append(pltpu.async_copy(
                        vrows.at[pl.ds(j * CW, CW)],
                        sh_den2.at[dstv.at[j]], semB, add=True))
                for d in descs2:
                    d.wait()
                return cr

            lax.fori_loop(lo, hi, macro2, 0)

        plsc.subcore_barrier()

        @pl.when(s < 8)
        def _():
            pltpu.sync_copy(sh_den2.at[pl.ds(s * 1280, 1280)],
                            den2_hbm.at[pl.ds(c * NP + s * 1280, 1280)])

    return passCD


_EM = 128
_EMC = E // _EM
_EK = _EM // CW


def _make_passE(t_tar):
    nsrc = t_tar + 1
    out_type = jax.ShapeDtypeStruct((2 * NP, HID), F32)
    scratch = [
        pltpu.VMEM((2, _EM), I32),
        pltpu.VMEM((1, CW), I32),
        pltpu.VMEM((1, CW), I32),
        pltpu.VMEM((2, _EM * HG), F32),
        pltpu.VMEM((_EM * HG,), F32),
        pltpu.VMEM((_EM, 16), F32),
        pltpu.VMEM((_EM, HID), F32),
        pltpu.VMEM((CW, HID), F32),
        pltpu.VMEM_SHARED((NP, HID), F32),
        pltpu.SemaphoreType.DMA,
        pltpu.SemaphoreType.DMA,
        pltpu.SemaphoreType.DMA,
    ]

    @functools.partial(
        pl.kernel, out_type=out_type, mesh=_mesh(), scratch_types=scratch,
        compiler_params=pltpu.CompilerParams(use_tc_tiling_on_sc=False))
    def passE(edges1d, ve_hbm, ra_hbm, den2_hbm, zacc, agg_hbm,
              dflat, dstv, div, raf, w2f, den2rows, ve, wv, sh_acc,
              semA, semB, semW):
        c = lax.axis_index("c")
        s = lax.axis_index("s")

        @pl.when(s < 5)
        def _():
            pltpu.sync_copy(zacc, sh_acc.at[pl.ds(s * 2048, 2048)])

        plsc.subcore_barrier()

        iota = _iota16()
        cidx = [(iota * 0) + kk for kk in range(16)]
        lo, hi = _span(s, _EMC)
        for ts in range(nsrc):
            rabase = (c * nsrc + ts) * E * HG
            ebase = (ts * 2 + 1) * E

            def prefetch(m, b, ts=ts, rabase=rabase, ebase=ebase):
                pltpu.async_copy(edges1d.at[pl.ds(ebase + m * _EM, _EM)],
                                 dflat.at[b], semA)
                pltpu.async_copy(
                    ra_hbm.at[pl.ds(rabase + m * _EM * HG, _EM * HG)],
                    raf.at[b], semA)

            def wait_prefetch(m, b, ts=ts, rabase=rabase, ebase=ebase):
                pltpu.make_async_copy(
                    edges1d.at[pl.ds(ebase + m * _EM, _EM)],
                    dflat.at[b], semA).wait()
                pltpu.make_async_copy(
                    ra_hbm.at[pl.ds(rabase + m * _EM * HG, _EM * HG)],
                    raf.at[b], semA).wait()

            prefetch(lo, lo & 1)

            def macro(m, cr, ts=ts, prefetch=prefetch,
                      wait_prefetch=wait_prefetch):
                b = m & 1
                wait_prefetch(m, b)
                for i in range(CW // 16):
                    w = dflat[b, pl.ds(i * 16, 16)]
                    dstv[0, pl.ds(i * 16, 16)] = w
                    div[0, pl.ds(i * 16, 16)] = w + c * NP
                d1 = pltpu.async_copy(den2_hbm.at[div.at[0]], den2rows, semB)
                d2 = pltpu.async_copy(
                    ve_hbm.at[pl.ds(ts * E + m * _EM, _EM)], ve, semB)

                @pl.when(m + 1 < hi)
                def _():
                    prefetch(m + 1, 1 - b)

                d1.wait()
                d2.wait()

                def quad(q, qcr):
                    sl = pl.ds(q * 16, 16)
                    den16 = _merge_den(den2rows, q)
                    w2f[sl] = jnp.exp(-raf[b, sl]) / (den16 + 1e-16)
                    return qcr

                lax.fori_loop(0, _EM // 4, quad, 0)

                @pl.when(m > lo)
                def _():
                    pltpu.make_async_copy(
                        ve_hbm.at[pl.ds(0, CW)], wv, semW).wait()

                def quad2(q2, qcr):
                    rwin = raf[b, pl.ds(q2 * 16, 16)]
                    wwin = w2f[pl.ds(q2 * 16, 16)]
                    for j2 in range(4):
                        e = q2 * 4 + j2
                        for h in range(HG):
                            vv = ve[e, pl.ds(c * HW + h * DK, DK)]
                            w1s = jnp.take(rwin, cidx[j2 * 4 + h])
                            w2s = jnp.take(wwin, cidx[j2 * 4 + h])
                            wv[e, pl.ds(h * DK, DK)] = vv * w1s
                            wv[e, pl.ds(HW + h * DK, DK)] = vv * w2s
                    return qcr

                lax.fori_loop(0, CW // 4, quad2, 0)
                pltpu.async_copy(wv, sh_acc.at[dstv.at[0]], semW, add=True)
                return cr

            lax.fori_loop(lo, hi, macro, 0)
            pltpu.make_async_copy(ve_hbm.at[pl.ds(0, CW)], wv, semW).wait()

        plsc.subcore_barrier()

        @pl.when(s < 5)
        def _():
            pltpu.sync_copy(sh_acc.at[pl.ds(s * 2048, 2048)],
                            agg_hbm.at[pl.ds(c * NP + s * 2048, 2048)])

    return passE


_gatherA = _make_gatherA()
_passCD = [_make_passCD(t) for t in range(T)]
_passE = [_make_passE(t) for t in range(T)]

_SEL = np.zeros((2, HID, HG), dtype=np.float32)
for _h in range(H):
    _SEL[_h // HG, _h * DK:(_h + 1) * DK, _h % HG] = 1.0


def kernel(x_list, edge_index_list, Wq, bq, Wk, bk, Wv, bv,
           ln_g, ln_b, W1, b1, W2, b2):
    xf = x_list.reshape(T * N, D_IN)
    q, k, v = _proj(xf, Wq, Wk, Wv,
                    bq.reshape(1, HID), bk.reshape(1, HID), bv.reshape(1, HID))

    qtab, ktab, vtab = q, k, v
    edges1d = edge_index_list.reshape(T * 2 * E)

    qe, ke, ve = _gatherA(qtab, ktab, vtab, edges1d)
    e1 = _att(qe, ke, jnp.asarray(_SEL)).reshape(12 * E * HG)

    zden = jnp.zeros((1280, 16), F32)
    zacc = jnp.zeros((2048, HID), F32)

    ys = []
    for t_tar in range(T):
        ra, den2 = _passCD[t_tar](edges1d, e1, zden)
        agg = _passE[t_tar](edges1d, ve, ra, den2, zacc)
        a0, a1 = agg[:N], agg[NP:NP + N]
        causal_hat = jnp.concatenate([a0[:, :HW], a1[:, :HW]], axis=1)
        spurious_hat = jnp.concatenate([a0[:, HW:], a1[:, HW:]], axis=1)
        ys.append(causal_hat + x_list[t_tar])
        ys.append(spurious_hat)

    y = jnp.stack(ys).reshape(2 * T * N, HID)
    f = _ffn(y, ln_g.reshape(1, HID), ln_b.reshape(1, HID),
             W1, b1.reshape(1, 2 * HID), W2, b2.reshape(1, HID))
    f = f.reshape(T, 2, N, HID)
    cs, ss = f[:, 0], f[:, 1]
    return cs + ss, cs, ss
